# Initial kernel scaffold; baseline (speedup 1.0000x reference)
import jax
import jax.numpy as jnp
from jax.experimental import pallas as pl


def kernel(x, eps, p0, p1, p2, p3, p4, p5, p6, p7, p8, p9, p10, p11, p12, p13, p14, p15, p16, p17, p18, p19, p20, p21, p22, p23, p24, p25, p26, p27, p28, p29, p30, p31, p32, p33, p34, p35, p36, p37, p38, p39, p40, p41, p42, p43, p44, p45, p46, p47, p48, p49, p50, p51, p52, p53, p54, p55, p56, p57, p58, p59, p60, p61, p62, p63, p64, p65, p66, p67, p68, p69, p70, p71, p72, p73, p74, p75, p76, p77, p78, p79, p80, p81, p82, p83, p84, p85, p86, p87, p88, p89, p90, p91, p92, p93, p94, p95, p96, p97, p98, p99, p100, p101, p102, p103, p104, p105, p106, p107, p108, p109, p110, p111, p112, p113, p114, p115, p116, p117, p118, p119, p120, p121, p122, p123, p124, p125, p126, p127, p128, p129, p130, p131, p132, p133, p134, p135, p136, p137, p138, p139, p140, p141, p142, p143, p144, p145, p146, p147, p148, p149, p150, p151, p152, p153, p154, p155, p156, p157, p158, p159, p160, p161, p162, p163, p164, p165, p166, p167, p168, p169, p170, p171, p172, p173, p174, p175, p176, p177, p178, p179, p180, p181):
    raise NotImplementedError("write your pallas kernel here")


# speedup vs baseline: 1.4188x; 1.4188x over previous
import functools

import jax
import jax.numpy as jnp
from jax.experimental import pallas as pl
from jax.experimental.pallas import tpu as pltpu

_LAT = 64
_NCLS = 3
_SLOPE = 0.01
_CROP_R = (150, 234)
_CROP_C = (24, 360)

_BLOCK_CFG = [
    (1, False, False),
    (2, True, False), (1, True, True),
    (2, True, False), (1, True, True), (1, True, True),
    (2, True, False), (1, True, True), (1, True, True), (1, True, True),
    (1, True, False), (1, True, True), (1, True, True),
    (2, True, False), (1, True, True), (1, True, True),
    (1, True, False),
]


def _ru(x, m):
    return (x + m - 1) // m * m


def _act(y, act):
    if act == "relu6":
        return jnp.clip(y, 0.0, 6.0)
    if act == "leaky":
        return jnp.where(y >= 0.0, y, _SLOPE * y)
    if act == "relu":
        return jnp.maximum(y, 0.0)
    return y


def _mm_body(*refs, act, has_res):
    if has_res:
        a_ref, b_ref, s_ref, c_ref, r_ref, o_ref = refs
    else:
        a_ref, b_ref, s_ref, c_ref, o_ref = refs
    y = jnp.dot(a_ref[...], b_ref[...], preferred_element_type=jnp.float32)
    y = _act(y * s_ref[...] + c_ref[...], act)
    if has_res:
        y = y + r_ref[...].astype(jnp.float32)
    o_ref[...] = y.astype(o_ref.dtype)


def _mm_fused(a, b, scale, bias, act="linear", residual=None,
              out_dtype=jnp.bfloat16):
    M, K = a.shape
    N = b.shape[1]
    has_res = residual is not None
    per_row = 2 * K + N * (4 if out_dtype == jnp.float32 else 2)
    if has_res:
        per_row += 2 * N
    bm = max(8, min(4096, (2 * 1024 * 1024 // per_row) // 8 * 8))
    bm = min(bm, _ru(M, 8))
    Mp = _ru(M, bm)
    if Mp != M:
        a = jnp.pad(a, ((0, Mp - M), (0, 0)))
        if has_res:
            residual = jnp.pad(residual, ((0, Mp - M), (0, 0)))
    specs = [
        pl.BlockSpec((bm, K), lambda i: (i, 0)),
        pl.BlockSpec((K, N), lambda i: (0, 0)),
        pl.BlockSpec((1, N), lambda i: (0, 0)),
        pl.BlockSpec((1, N), lambda i: (0, 0)),
    ]
    ops = [a, b, scale, bias]
    if has_res:
        specs.append(pl.BlockSpec((bm, N), lambda i: (i, 0)))
        ops.append(residual)
    out = pl.pallas_call(
        functools.partial(_mm_body, act=act, has_res=has_res),
        out_shape=jax.ShapeDtypeStruct((Mp, N), out_dtype),
        grid=(Mp // bm,),
        in_specs=specs,
        out_specs=pl.BlockSpec((bm, N), lambda i: (i, 0)),
        compiler_params=pltpu.CompilerParams(
            dimension_semantics=("parallel",)),
    )(*ops)
    return out[:M] if Mp != M else out


def _im2col(x, kh, kw, stride, padding):
    if padding:
        x = jnp.pad(x, ((0, 0), (padding, padding), (padding, padding), (0, 0)))
    N, H, W, C = x.shape
    Ho = (H - kh) // stride + 1
    Wo = (W - kw) // stride + 1
    cols = [x[:, dy:dy + stride * Ho:stride, dx:dx + stride * Wo:stride, :]
            for dy in range(kh) for dx in range(kw)]
    patches = jnp.stack(cols, axis=3)
    return patches.reshape(N * Ho * Wo, kh * kw * C), (N, Ho, Wo)


def _conv(x, w, kh, kw, stride, padding, scale, bias, act,
          residual=None, out_dtype=jnp.bfloat16):
    x = x.astype(jnp.bfloat16)
    N, H, W, C = x.shape
    Cout = w.shape[1]
    if kh == 1 and kw == 1 and stride == 1 and padding == 0:
        patches, (N, Ho, Wo) = x.reshape(N * H * W, C), (N, H, W)
    else:
        patches, (N, Ho, Wo) = _im2col(x, kh, kw, stride, padding)
    res = residual.reshape(N * Ho * Wo, Cout) if residual is not None else None
    y = _mm_fused(patches, w, scale, bias, act, residual=res,
                  out_dtype=out_dtype)
    return y.reshape(N, Ho, Wo, Cout)


def _dw_s1_body(x_ref, w_ref, s_ref, c_ref, o_ref, *, Ho, Wo):
    xp = x_ref[0]
    acc = None
    for dy in range(3):
        for dx in range(3):
            t = xp[dy:dy + Ho, dx:dx + Wo, :].astype(jnp.float32)
            term = t * w_ref[3 * dy + dx]
            acc = term if acc is None else acc + term
    y = jnp.clip(acc * s_ref[...] + c_ref[...], 0.0, 6.0)
    o_ref[0] = y.astype(jnp.bfloat16)


def _dw_s2_body(p00, p01, p10, p11, w_ref, s_ref, c_ref, o_ref, *, Ho, Wo):
    phases = ((p00, p01), (p10, p11))
    acc = None
    for dy in range(3):
        for dx in range(3):
            ph = phases[dy % 2][dx % 2]
            oy, ox = dy // 2, dx // 2
            t = ph[0][oy:oy + Ho, ox:ox + Wo, :].astype(jnp.float32)
            term = t * w_ref[3 * dy + dx]
            acc = term if acc is None else acc + term
    y = jnp.clip(acc * s_ref[...] + c_ref[...], 0.0, 6.0)
    o_ref[0] = y.astype(jnp.bfloat16)


def _depthwise(x, w9, scale, bias, stride):
    N, H, W, C = x.shape
    w = w9.reshape(9, C)
    xp = jnp.pad(x, ((0, 0), (1, 1), (1, 1), (0, 0)))
    wspec = pl.BlockSpec((9, C), lambda i: (0, 0))
    sspec = pl.BlockSpec((1, C), lambda i: (0, 0))
    if stride == 1:
        Ho, Wo = H, W
        out = pl.pallas_call(
            functools.partial(_dw_s1_body, Ho=Ho, Wo=Wo),
            out_shape=jax.ShapeDtypeStruct((N, Ho, Wo, C), jnp.bfloat16),
            grid=(N,),
            in_specs=[pl.BlockSpec((1, H + 2, W + 2, C), lambda i: (i, 0, 0, 0)),
                      wspec, sspec, sspec],
            out_specs=pl.BlockSpec((1, Ho, Wo, C), lambda i: (i, 0, 0, 0)),
            compiler_params=pltpu.CompilerParams(
                dimension_semantics=("parallel",)),
        )(xp, w, scale, bias)
    else:
        Ho = (H + 2 - 3) // 2 + 1
        Wo = (W + 2 - 3) // 2 + 1
        ph = [xp[:, q::2, r::2, :] for q in range(2) for r in range(2)]
        Pr, Pc = ph[0].shape[1], ph[0].shape[2]
        pspec = pl.BlockSpec((1, Pr, Pc, C), lambda i: (i, 0, 0, 0))
        out = pl.pallas_call(
            functools.partial(_dw_s2_body, Ho=Ho, Wo=Wo),
            out_shape=jax.ShapeDtypeStruct((N, Ho, Wo, C), jnp.bfloat16),
            grid=(N,),
            in_specs=[pspec, pspec, pspec, pspec, wspec, sspec, sspec],
            out_specs=pl.BlockSpec((1, Ho, Wo, C), lambda i: (i, 0, 0, 0)),
            compiler_params=pltpu.CompilerParams(
                dimension_semantics=("parallel",)),
        )(*ph, w, scale, bias)
    return out


def _pool_body(x_ref, o_ref):
    m = jnp.mean(x_ref[...].astype(jnp.float32), axis=1)
    o_ref[...] = jnp.maximum(m, 0.0)


def _avgpool_relu(x):
    N, HW, C = x.shape
    return pl.pallas_call(
        _pool_body,
        out_shape=jax.ShapeDtypeStruct((N, C), jnp.float32),
    )(x)


def _reparam_body(mu_ref, lv_ref, eps_ref, o_ref):
    o_ref[...] = eps_ref[...] * jnp.exp(0.5 * lv_ref[...]) + mu_ref[...]


def _reparam(mu, log_var, eps):
    return pl.pallas_call(
        _reparam_body,
        out_shape=jax.ShapeDtypeStruct(mu.shape, jnp.float32),
    )(mu, log_var, eps)


def _cdiv(a, b):
    return -(-a // b)


def _convt_out(hin, k, s, op):
    return (hin - 1) * s + k + op


def _convt_need(win, k, s, hin):
    a, b = win
    d_last = (hin - 1) * s
    d_lo = max(a - (k - 1), 0)
    d_hi = min(b - 1, d_last)
    i_lo = max(0, min(_cdiv(d_lo, s), hin - 1))
    i_hi = max(i_lo, min(d_hi // s, hin - 1))
    return (i_lo, i_hi + 1)


def _convt_window(x, in_off, w, k, s, hin, win, owr, owc, scale, bias):
    N, xr, xc, C = x.shape
    ar, br = owr
    ac, bc = owc
    ir_lo, ir_hi = _convt_need(owr, k, s, hin)
    ic_lo, ic_hi = _convt_need(owc, k, s, win)
    xs = x[:, ir_lo - in_off[0]:ir_hi - in_off[0],
           ic_lo - in_off[1]:ic_hi - in_off[1], :].astype(jnp.bfloat16)
    nr, nc = ir_hi - ir_lo, ic_hi - ic_lo
    Lr = (br - ar) + k - 1
    Lc = (bc - ac) + k - 1
    sr = ir_lo * s + (k - 1) - ar
    sc = ic_lo * s + (k - 1) - ac
    buf = jnp.zeros((N, Lr, Lc, C), jnp.bfloat16)
    buf = buf.at[:, sr:sr + (nr - 1) * s + 1:s,
                 sc:sc + (nc - 1) * s + 1:s, :].set(xs)
    return _conv(buf, w, k, k, 1, 0, scale, bias, "leaky")


def _encode(x_nhwc, stem, blocks, head, post, fc_mu, fc_var):
    stem_w, stem_s, stem_b = stem
    x = _conv(x_nhwc, stem_w, 3, 3, 2, 1, stem_s, stem_b, "relu6")
    for blk, (stride, has_exp, use_res) in zip(blocks, _BLOCK_CFG):
        inp = x
        h = x
        if has_exp:
            h = _conv(h, blk["exp_w"], 1, 1, 1, 0,
                      blk["exp_s"], blk["exp_b"], "relu6")
        h = _depthwise(h, blk["dw_w"], blk["dw_s"], blk["dw_b"], stride)
        x = _conv(h, blk["proj_w"], 1, 1, 1, 0,
                  blk["proj_s"], blk["proj_b"], "linear",
                  residual=inp if use_res else None)
    head_w, head_s, head_b = head
    x = _conv(x, head_w, 1, 1, 1, 0, head_s, head_b, "relu6")
    N, H, W, C = x.shape
    feat = _avgpool_relu(x.reshape(N, H * W, C))
    ones = jnp.ones((1, _LAT), jnp.float32)
    r = _mm_fused(feat.astype(jnp.bfloat16), post[0], ones, post[1],
                  "leaky", out_dtype=jnp.float32)
    mu = _mm_fused(r.astype(jnp.bfloat16), fc_mu[0], ones, fc_mu[1],
                   "linear", out_dtype=jnp.float32)
    log_var = _mm_fused(r.astype(jnp.bfloat16), fc_var[0], ones, fc_var[1],
                        "linear", out_dtype=jnp.float32)
    return mu, log_var


_DEC_CFG = [
    (_LAT // 4, 64, 5, 2, 0),
    (64, 64, 3, 2, 0),
    (64, 32, 3, 2, 0),
    (32, 16, 5, 3, 0),
    (16, 8, 3, 2, 0),
    (8, 8, 3, 2, 1),
]


def _decode(z, dec, final_w, final_b):
    N = z.shape[0]
    x = jnp.transpose(z.reshape(N, _LAT // 4, 2, 2), (0, 2, 3, 1))
    sizes = [(2, 2)]
    for (_, _, k, s, op) in _DEC_CFG:
        h, w = sizes[-1]
        sizes.append((_convt_out(h, k, s, op), _convt_out(w, k, s, op)))
    nly = len(_DEC_CFG)
    wins = [None] * nly
    wins[-1] = ((_CROP_R[0] - 1, _CROP_R[1] + 1),
                (_CROP_C[0] - 1, _CROP_C[1] + 1))
    for li in range(nly - 1, 0, -1):
        (_, _, k, s, _) = _DEC_CFG[li]
        hin, win = sizes[li]
        wins[li - 1] = (_convt_need(wins[li][0], k, s, hin),
                        _convt_need(wins[li][1], k, s, win))
    in_off = (0, 0)
    for li, ((_, _, k, s, _), ly) in enumerate(zip(_DEC_CFG, dec)):
        hin, win = sizes[li]
        owr, owc = wins[li]
        x = _convt_window(x, in_off, ly["w"], k, s, hin, win, owr, owc,
                          ly["scale"], ly["bias"])
        in_off = (owr[0], owc[0])
    ones = jnp.ones((1, _NCLS), jnp.float32)
    y = _conv(x, final_w, 3, 3, 1, 0, ones, final_b, "linear",
              out_dtype=jnp.float32)
    return jnp.transpose(y, (0, 3, 1, 2))


def _unpack(params):
    it = iter(params[0:150])
    blocks = []
    for (stride, has_exp, use_res) in _BLOCK_CFG:
        blk = {}
        blk["dw_s"], blk["dw_b"], blk["dw_w"] = next(it), next(it), next(it)
        if has_exp:
            blk["exp_s"], blk["exp_b"], blk["exp_w"] = \
                next(it), next(it), next(it)
        blk["proj_s"], blk["proj_b"], blk["proj_w"] = \
            next(it), next(it), next(it)
        blocks.append(blk)
    dec = []
    for li in range(6):
        b, s, w = params[150 + 3 * li: 153 + 3 * li]
        dec.append({"bias": b, "scale": s, "w": w})
    (final_b, final_w, head_s, head_b, head_w, mu_b, mu_w,
     post_b, post_w, stem_s, stem_b, stem_w, var_b, var_w) = params[168:182]
    return dict(
        blocks=blocks, dec=dec, final_b=final_b, final_w=final_w,
        head=(head_w, head_s, head_b), fc_mu=(mu_w, mu_b),
        post=(post_w, post_b), stem=(stem_w, stem_s, stem_b),
        fc_var=(var_w, var_b))


def kernel(x, eps, *params):
    P = _unpack(params)
    x_nhwc = jnp.transpose(x, (0, 2, 3, 1)).astype(jnp.float32)
    mu, log_var = _encode(x_nhwc, P["stem"], P["blocks"], P["head"],
                          P["post"], P["fc_mu"], P["fc_var"])
    z = _reparam(mu, log_var, eps)
    dec_params = [dict(w=d["w"], scale=d["scale"], bias=d["bias"])
                  for d in P["dec"]]
    recons = _decode(z, dec_params, P["final_w"], P["final_b"])
    return recons, x, mu, log_var


# speedup vs baseline: 4.4330x; 3.1245x over previous
import functools

import jax
import jax.numpy as jnp
from jax.experimental import pallas as pl
from jax.experimental.pallas import tpu as pltpu

_LAT = 64
_NCLS = 3
_SLOPE = 0.01
_CROP_R = (150, 234)
_CROP_C = (24, 360)

_BLOCK_CFG = [
    (1, False, False),
    (2, True, False), (1, True, True),
    (2, True, False), (1, True, True), (1, True, True),
    (2, True, False), (1, True, True), (1, True, True), (1, True, True),
    (1, True, False), (1, True, True), (1, True, True),
    (2, True, False), (1, True, True), (1, True, True),
    (1, True, False),
]


def _ru(x, m):
    return (x + m - 1) // m * m


def _act(y, act):
    if act == "relu6":
        return jnp.clip(y, 0.0, 6.0)
    if act == "leaky":
        return jnp.where(y >= 0.0, y, _SLOPE * y)
    if act == "relu":
        return jnp.maximum(y, 0.0)
    return y


def _mm_body(*refs, act, has_res):
    if has_res:
        a_ref, b_ref, s_ref, c_ref, r_ref, o_ref = refs
    else:
        a_ref, b_ref, s_ref, c_ref, o_ref = refs
    y = jnp.dot(a_ref[...], b_ref[...], preferred_element_type=jnp.float32)
    y = _act(y * s_ref[...] + c_ref[...], act)
    if has_res:
        y = y + r_ref[...].astype(jnp.float32)
    o_ref[...] = y.astype(o_ref.dtype)


def _mm_fused(a, b, scale, bias, act="linear", residual=None,
              out_dtype=jnp.bfloat16):
    M, K = a.shape
    N = b.shape[1]
    has_res = residual is not None
    per_row = 2 * K + N * (4 if out_dtype == jnp.float32 else 2)
    if has_res:
        per_row += 2 * N
    bm = max(8, min(4096, (2 * 1024 * 1024 // per_row) // 8 * 8))
    bm = min(bm, _ru(M, 8))
    Mp = _ru(M, bm)
    if Mp != M:
        a = jnp.pad(a, ((0, Mp - M), (0, 0)))
        if has_res:
            residual = jnp.pad(residual, ((0, Mp - M), (0, 0)))
    specs = [
        pl.BlockSpec((bm, K), lambda i: (i, 0)),
        pl.BlockSpec((K, N), lambda i: (0, 0)),
        pl.BlockSpec((1, N), lambda i: (0, 0)),
        pl.BlockSpec((1, N), lambda i: (0, 0)),
    ]
    ops = [a, b, scale, bias]
    if has_res:
        specs.append(pl.BlockSpec((bm, N), lambda i: (i, 0)))
        ops.append(residual)
    out = pl.pallas_call(
        functools.partial(_mm_body, act=act, has_res=has_res),
        out_shape=jax.ShapeDtypeStruct((Mp, N), out_dtype),
        grid=(Mp // bm,),
        in_specs=specs,
        out_specs=pl.BlockSpec((bm, N), lambda i: (i, 0)),
        compiler_params=pltpu.CompilerParams(
            dimension_semantics=("parallel",)),
    )(*ops)
    return out[:M] if Mp != M else out


def _im2col(x, kh, kw, stride, padding):
    if padding:
        x = jnp.pad(x, ((0, 0), (padding, padding), (padding, padding), (0, 0)))
    N, H, W, C = x.shape
    Ho = (H - kh) // stride + 1
    Wo = (W - kw) // stride + 1
    cols = [x[:, dy:dy + stride * Ho:stride, dx:dx + stride * Wo:stride, :]
            for dy in range(kh) for dx in range(kw)]
    patches = jnp.stack(cols, axis=3)
    return patches.reshape(N * Ho * Wo, kh * kw * C), (N, Ho, Wo)


def _conv(x, w, kh, kw, stride, padding, scale, bias, act,
          residual=None, out_dtype=jnp.bfloat16):
    x = x.astype(jnp.bfloat16)
    N, H, W, C = x.shape
    Cout = w.shape[1]
    if kh == 1 and kw == 1 and stride == 1 and padding == 0:
        patches, (N, Ho, Wo) = x.reshape(N * H * W, C), (N, H, W)
    else:
        patches, (N, Ho, Wo) = _im2col(x, kh, kw, stride, padding)
    res = residual.reshape(N * Ho * Wo, Cout) if residual is not None else None
    y = _mm_fused(patches, w, scale, bias, act, residual=res,
                  out_dtype=out_dtype)
    return y.reshape(N, Ho, Wo, Cout)


def _dw_s1_body(x_ref, w_ref, s_ref, c_ref, o_ref, *, Ho, Wo):
    xp = x_ref[0]
    acc = None
    for dy in range(3):
        for dx in range(3):
            t = xp[dy:dy + Ho, dx:dx + Wo, :].astype(jnp.float32)
            term = t * w_ref[3 * dy + dx]
            acc = term if acc is None else acc + term
    y = jnp.clip(acc * s_ref[...] + c_ref[...], 0.0, 6.0)
    o_ref[0] = y.astype(jnp.bfloat16)


def _dw_s2_body(p00, p01, p10, p11, w_ref, s_ref, c_ref, o_ref, *, Ho, Wo):
    phases = ((p00, p01), (p10, p11))
    acc = None
    for dy in range(3):
        for dx in range(3):
            ph = phases[dy % 2][dx % 2]
            oy, ox = dy // 2, dx // 2
            t = ph[0][oy:oy + Ho, ox:ox + Wo, :].astype(jnp.float32)
            term = t * w_ref[3 * dy + dx]
            acc = term if acc is None else acc + term
    y = jnp.clip(acc * s_ref[...] + c_ref[...], 0.0, 6.0)
    o_ref[0] = y.astype(jnp.bfloat16)


def _depthwise(x, w9, scale, bias, stride):
    N, H, W, C = x.shape
    w = w9.reshape(9, C)
    xp = jnp.pad(x, ((0, 0), (1, 1), (1, 1), (0, 0)))
    wspec = pl.BlockSpec((9, C), lambda i: (0, 0))
    sspec = pl.BlockSpec((1, C), lambda i: (0, 0))
    if stride == 1:
        Ho, Wo = H, W
        out = pl.pallas_call(
            functools.partial(_dw_s1_body, Ho=Ho, Wo=Wo),
            out_shape=jax.ShapeDtypeStruct((N, Ho, Wo, C), jnp.bfloat16),
            grid=(N,),
            in_specs=[pl.BlockSpec((1, H + 2, W + 2, C), lambda i: (i, 0, 0, 0)),
                      wspec, sspec, sspec],
            out_specs=pl.BlockSpec((1, Ho, Wo, C), lambda i: (i, 0, 0, 0)),
            compiler_params=pltpu.CompilerParams(
                dimension_semantics=("parallel",)),
        )(xp, w, scale, bias)
    else:
        Ho = (H + 2 - 3) // 2 + 1
        Wo = (W + 2 - 3) // 2 + 1
        ph = [xp[:, q::2, r::2, :] for q in range(2) for r in range(2)]
        Pr, Pc = ph[0].shape[1], ph[0].shape[2]
        pspec = pl.BlockSpec((1, Pr, Pc, C), lambda i: (i, 0, 0, 0))
        out = pl.pallas_call(
            functools.partial(_dw_s2_body, Ho=Ho, Wo=Wo),
            out_shape=jax.ShapeDtypeStruct((N, Ho, Wo, C), jnp.bfloat16),
            grid=(N,),
            in_specs=[pspec, pspec, pspec, pspec, wspec, sspec, sspec],
            out_specs=pl.BlockSpec((1, Ho, Wo, C), lambda i: (i, 0, 0, 0)),
            compiler_params=pltpu.CompilerParams(
                dimension_semantics=("parallel",)),
        )(*ph, w, scale, bias)
    return out


def _convd_body(m_ref, h_ref, w_ref, s_ref, c_ref, o_ref, *,
                k, TR, Wo, act):
    C = m_ref.shape[3]
    xw = jnp.concatenate([m_ref[0], h_ref[0]], axis=0)
    acc = None
    for dy in range(k):
        for dx in range(k):
            a = xw[dy:dy + TR, dx:dx + Wo, :].reshape(TR * Wo, C)
            t = (dy * k + dx) * C
            y = jnp.dot(a, w_ref[t:t + C, :],
                        preferred_element_type=jnp.float32)
            acc = y if acc is None else acc + y
    y = _act(acc * s_ref[...] + c_ref[...], act)
    o_ref[0] = y.reshape(TR, Wo, o_ref.shape[3]).astype(o_ref.dtype)


def _conv_direct(x, w, k, scale, bias, act, out_dtype=jnp.bfloat16):
    x = x.astype(jnp.bfloat16)
    N, H, W, C = x.shape
    Cout = w.shape[1]
    Ho, Wo = H - k + 1, W - k + 1
    Wop = _ru(Wo, 16)
    TR = min(32, _ru(Ho, 8))
    nt = -(-Ho // TR)
    xp = jnp.pad(x, ((0, 0), (0, TR * nt + 8 - H), (0, Wop + k - 1 - W),
                     (0, 0)))
    Lc = Wop + k - 1
    out = pl.pallas_call(
        functools.partial(_convd_body, k=k, TR=TR, Wo=Wop, act=act),
        out_shape=jax.ShapeDtypeStruct((N, TR * nt, Wop, Cout), out_dtype),
        grid=(N, nt),
        in_specs=[
            pl.BlockSpec((1, TR, Lc, C), lambda i, j: (i, j, 0, 0)),
            pl.BlockSpec((1, 8, Lc, C),
                         lambda i, j: (i, (j * TR + TR) // 8, 0, 0)),
            pl.BlockSpec((k * k * C, Cout), lambda i, j: (0, 0)),
            pl.BlockSpec((1, Cout), lambda i, j: (0, 0)),
            pl.BlockSpec((1, Cout), lambda i, j: (0, 0)),
        ],
        out_specs=pl.BlockSpec((1, TR, Wop, Cout), lambda i, j: (i, j, 0, 0)),
        compiler_params=pltpu.CompilerParams(
            dimension_semantics=("parallel", "parallel")),
    )(xp, xp, w, scale, bias)
    return out[:, :Ho, :Wo, :]


def _pool_body(x_ref, o_ref):
    m = jnp.mean(x_ref[...].astype(jnp.float32), axis=1)
    o_ref[...] = jnp.maximum(m, 0.0)


def _avgpool_relu(x):
    N, HW, C = x.shape
    return pl.pallas_call(
        _pool_body,
        out_shape=jax.ShapeDtypeStruct((N, C), jnp.float32),
    )(x)


def _reparam_body(mu_ref, lv_ref, eps_ref, o_ref):
    o_ref[...] = eps_ref[...] * jnp.exp(0.5 * lv_ref[...]) + mu_ref[...]


def _reparam(mu, log_var, eps):
    return pl.pallas_call(
        _reparam_body,
        out_shape=jax.ShapeDtypeStruct(mu.shape, jnp.float32),
    )(mu, log_var, eps)


def _cdiv(a, b):
    return -(-a // b)


def _convt_out(hin, k, s, op):
    return (hin - 1) * s + k + op


def _convt_need(win, k, s, hin):
    a, b = win
    d_last = (hin - 1) * s
    d_lo = max(a - (k - 1), 0)
    d_hi = min(b - 1, d_last)
    i_lo = max(0, min(_cdiv(d_lo, s), hin - 1))
    i_hi = max(i_lo, min(d_hi // s, hin - 1))
    return (i_lo, i_hi + 1)


def _convt_window(x, in_off, w, k, s, hin, win, owr, owc, scale, bias):
    N, xr, xc, C = x.shape
    ar, br = owr
    ac, bc = owc
    ir_lo, ir_hi = _convt_need(owr, k, s, hin)
    ic_lo, ic_hi = _convt_need(owc, k, s, win)
    xs = x[:, ir_lo - in_off[0]:ir_hi - in_off[0],
           ic_lo - in_off[1]:ic_hi - in_off[1], :].astype(jnp.bfloat16)
    nr, nc = ir_hi - ir_lo, ic_hi - ic_lo
    Lr = (br - ar) + k - 1
    Lc = (bc - ac) + k - 1
    sr = ir_lo * s + (k - 1) - ar
    sc = ic_lo * s + (k - 1) - ac
    buf = jnp.zeros((N, Lr, Lc, C), jnp.bfloat16)
    buf = buf.at[:, sr:sr + (nr - 1) * s + 1:s,
                 sc:sc + (nc - 1) * s + 1:s, :].set(xs)
    return _conv_direct(buf, w, k, scale, bias, "leaky")


def _encode(x_nhwc, stem, blocks, head, post, fc_mu, fc_var):
    stem_w, stem_s, stem_b = stem
    x = _conv(x_nhwc, stem_w, 3, 3, 2, 1, stem_s, stem_b, "relu6")
    for blk, (stride, has_exp, use_res) in zip(blocks, _BLOCK_CFG):
        inp = x
        h = x
        if has_exp:
            h = _conv(h, blk["exp_w"], 1, 1, 1, 0,
                      blk["exp_s"], blk["exp_b"], "relu6")
        h = _depthwise(h, blk["dw_w"], blk["dw_s"], blk["dw_b"], stride)
        x = _conv(h, blk["proj_w"], 1, 1, 1, 0,
                  blk["proj_s"], blk["proj_b"], "linear",
                  residual=inp if use_res else None)
    head_w, head_s, head_b = head
    x = _conv(x, head_w, 1, 1, 1, 0, head_s, head_b, "relu6")
    N, H, W, C = x.shape
    feat = _avgpool_relu(x.reshape(N, H * W, C))
    ones = jnp.ones((1, _LAT), jnp.float32)
    r = _mm_fused(feat.astype(jnp.bfloat16), post[0], ones, post[1],
                  "leaky", out_dtype=jnp.float32)
    mu = _mm_fused(r.astype(jnp.bfloat16), fc_mu[0], ones, fc_mu[1],
                   "linear", out_dtype=jnp.float32)
    log_var = _mm_fused(r.astype(jnp.bfloat16), fc_var[0], ones, fc_var[1],
                        "linear", out_dtype=jnp.float32)
    return mu, log_var


_DEC_CFG = [
    (_LAT // 4, 64, 5, 2, 0),
    (64, 64, 3, 2, 0),
    (64, 32, 3, 2, 0),
    (32, 16, 5, 3, 0),
    (16, 8, 3, 2, 0),
    (8, 8, 3, 2, 1),
]


def _decode(z, dec, final_w, final_b):
    N = z.shape[0]
    x = jnp.transpose(z.reshape(N, _LAT // 4, 2, 2), (0, 2, 3, 1))
    sizes = [(2, 2)]
    for (_, _, k, s, op) in _DEC_CFG:
        h, w = sizes[-1]
        sizes.append((_convt_out(h, k, s, op), _convt_out(w, k, s, op)))
    nly = len(_DEC_CFG)
    wins = [None] * nly
    wins[-1] = ((_CROP_R[0] - 1, _CROP_R[1] + 1),
                (_CROP_C[0] - 1, _CROP_C[1] + 1))
    for li in range(nly - 1, 0, -1):
        (_, _, k, s, _) = _DEC_CFG[li]
        hin, win = sizes[li]
        wins[li - 1] = (_convt_need(wins[li][0], k, s, hin),
                        _convt_need(wins[li][1], k, s, win))
    in_off = (0, 0)
    for li, ((_, _, k, s, _), ly) in enumerate(zip(_DEC_CFG, dec)):
        hin, win = sizes[li]
        owr, owc = wins[li]
        x = _convt_window(x, in_off, ly["w"], k, s, hin, win, owr, owc,
                          ly["scale"], ly["bias"])
        in_off = (owr[0], owc[0])
    ones = jnp.ones((1, _NCLS), jnp.float32)
    y = _conv_direct(x, final_w, 3, ones, final_b, "linear",
                     out_dtype=jnp.float32)
    return jnp.transpose(y, (0, 3, 1, 2))


def _unpack(params):
    it = iter(params[0:150])
    blocks = []
    for (stride, has_exp, use_res) in _BLOCK_CFG:
        blk = {}
        blk["dw_s"], blk["dw_b"], blk["dw_w"] = next(it), next(it), next(it)
        if has_exp:
            blk["exp_s"], blk["exp_b"], blk["exp_w"] = \
                next(it), next(it), next(it)
        blk["proj_s"], blk["proj_b"], blk["proj_w"] = \
            next(it), next(it), next(it)
        blocks.append(blk)
    dec = []
    for li in range(6):
        b, s, w = params[150 + 3 * li: 153 + 3 * li]
        dec.append({"bias": b, "scale": s, "w": w})
    (final_b, final_w, head_s, head_b, head_w, mu_b, mu_w,
     post_b, post_w, stem_s, stem_b, stem_w, var_b, var_w) = params[168:182]
    return dict(
        blocks=blocks, dec=dec, final_b=final_b, final_w=final_w,
        head=(head_w, head_s, head_b), fc_mu=(mu_w, mu_b),
        post=(post_w, post_b), stem=(stem_w, stem_s, stem_b),
        fc_var=(var_w, var_b))


def kernel(x, eps, *params):
    P = _unpack(params)
    x_nhwc = jnp.transpose(x, (0, 2, 3, 1)).astype(jnp.float32)
    mu, log_var = _encode(x_nhwc, P["stem"], P["blocks"], P["head"],
                          P["post"], P["fc_mu"], P["fc_var"])
    z = _reparam(mu, log_var, eps)
    dec_params = [dict(w=d["w"], scale=d["scale"], bias=d["bias"])
                  for d in P["dec"]]
    recons = _decode(z, dec_params, P["final_w"], P["final_b"])
    return recons, x, mu, log_var


# speedup vs baseline: 7.7836x; 1.7558x over previous
import functools

import jax
import jax.numpy as jnp
from jax.experimental import pallas as pl
from jax.experimental.pallas import tpu as pltpu

_LAT = 64
_NCLS = 3
_SLOPE = 0.01
_CROP_R = (150, 234)
_CROP_C = (24, 360)

_BLOCK_CFG = [
    (1, False, False),
    (2, True, False), (1, True, True),
    (2, True, False), (1, True, True), (1, True, True),
    (2, True, False), (1, True, True), (1, True, True), (1, True, True),
    (1, True, False), (1, True, True), (1, True, True),
    (2, True, False), (1, True, True), (1, True, True),
    (1, True, False),
]


def _ru(x, m):
    return (x + m - 1) // m * m


def _act(y, act):
    if act == "relu6":
        return jnp.clip(y, 0.0, 6.0)
    if act == "leaky":
        return jnp.where(y >= 0.0, y, _SLOPE * y)
    if act == "relu":
        return jnp.maximum(y, 0.0)
    return y


def _mm_body(*refs, act, has_res):
    if has_res:
        a_ref, b_ref, s_ref, c_ref, r_ref, o_ref = refs
    else:
        a_ref, b_ref, s_ref, c_ref, o_ref = refs
    y = jnp.dot(a_ref[...], b_ref[...], preferred_element_type=jnp.float32)
    y = _act(y * s_ref[...] + c_ref[...], act)
    if has_res:
        y = y + r_ref[...].astype(jnp.float32)
    o_ref[...] = y.astype(o_ref.dtype)


def _mm_fused(a, b, scale, bias, act="linear", residual=None,
              out_dtype=jnp.bfloat16):
    M, K = a.shape
    N = b.shape[1]
    has_res = residual is not None
    per_row = 2 * K + N * (4 if out_dtype == jnp.float32 else 2)
    if has_res:
        per_row += 2 * N
    bm = max(8, min(4096, (2 * 1024 * 1024 // per_row) // 8 * 8))
    bm = min(bm, _ru(M, 8))
    Mp = _ru(M, bm)
    if Mp != M:
        a = jnp.pad(a, ((0, Mp - M), (0, 0)))
        if has_res:
            residual = jnp.pad(residual, ((0, Mp - M), (0, 0)))
    specs = [
        pl.BlockSpec((bm, K), lambda i: (i, 0)),
        pl.BlockSpec((K, N), lambda i: (0, 0)),
        pl.BlockSpec((1, N), lambda i: (0, 0)),
        pl.BlockSpec((1, N), lambda i: (0, 0)),
    ]
    ops = [a, b, scale, bias]
    if has_res:
        specs.append(pl.BlockSpec((bm, N), lambda i: (i, 0)))
        ops.append(residual)
    out = pl.pallas_call(
        functools.partial(_mm_body, act=act, has_res=has_res),
        out_shape=jax.ShapeDtypeStruct((Mp, N), out_dtype),
        grid=(Mp // bm,),
        in_specs=specs,
        out_specs=pl.BlockSpec((bm, N), lambda i: (i, 0)),
        compiler_params=pltpu.CompilerParams(
            dimension_semantics=("parallel",)),
    )(*ops)
    return out[:M] if Mp != M else out


def _im2col(x, kh, kw, stride, padding):
    if padding:
        x = jnp.pad(x, ((0, 0), (padding, padding), (padding, padding), (0, 0)))
    N, H, W, C = x.shape
    Ho = (H - kh) // stride + 1
    Wo = (W - kw) // stride + 1
    cols = [x[:, dy:dy + stride * Ho:stride, dx:dx + stride * Wo:stride, :]
            for dy in range(kh) for dx in range(kw)]
    patches = jnp.stack(cols, axis=3)
    return patches.reshape(N * Ho * Wo, kh * kw * C), (N, Ho, Wo)


def _conv(x, w, kh, kw, stride, padding, scale, bias, act,
          residual=None, out_dtype=jnp.bfloat16):
    x = x.astype(jnp.bfloat16)
    N, H, W, C = x.shape
    Cout = w.shape[1]
    if kh == 1 and kw == 1 and stride == 1 and padding == 0:
        patches, (N, Ho, Wo) = x.reshape(N * H * W, C), (N, H, W)
    else:
        patches, (N, Ho, Wo) = _im2col(x, kh, kw, stride, padding)
    res = residual.reshape(N * Ho * Wo, Cout) if residual is not None else None
    y = _mm_fused(patches, w, scale, bias, act, residual=res,
                  out_dtype=out_dtype)
    return y.reshape(N, Ho, Wo, Cout)


def _dw_s1_body(x_ref, w_ref, s_ref, c_ref, o_ref, *, Ho, Wo):
    xp = x_ref[0]
    acc = None
    for dy in range(3):
        for dx in range(3):
            t = xp[dy:dy + Ho, dx:dx + Wo, :].astype(jnp.float32)
            term = t * w_ref[3 * dy + dx]
            acc = term if acc is None else acc + term
    y = jnp.clip(acc * s_ref[...] + c_ref[...], 0.0, 6.0)
    o_ref[0] = y.astype(jnp.bfloat16)


def _dw_s2_body(q0_ref, q1_ref, w_ref, s_ref, c_ref, o_ref, *, Ho, Wo, C):
    pb = (q0_ref[0, :, 0], q1_ref[0, :, 0])
    acc = None
    for dy in range(3):
        for dx in range(3):
            r = dx % 2
            ph = pb[dy % 2][:, :, r * C:(r + 1) * C]
            t = ph[dy // 2:dy // 2 + Ho, dx // 2:dx // 2 + Wo, :]
            term = t.astype(jnp.float32) * w_ref[3 * dy + dx]
            acc = term if acc is None else acc + term
    y = jnp.clip(acc * s_ref[...] + c_ref[...], 0.0, 6.0)
    o_ref[0] = y.astype(jnp.bfloat16)


def _depthwise(x, w9, scale, bias, stride):
    N, H, W, C = x.shape
    w = w9.reshape(9, C)
    xp = jnp.pad(x, ((0, 0), (1, 1), (1, 1), (0, 0)))
    wspec = pl.BlockSpec((9, C), lambda i: (0, 0))
    sspec = pl.BlockSpec((1, C), lambda i: (0, 0))
    if stride == 1:
        Ho, Wo = H, W
        out = pl.pallas_call(
            functools.partial(_dw_s1_body, Ho=Ho, Wo=Wo),
            out_shape=jax.ShapeDtypeStruct((N, Ho, Wo, C), jnp.bfloat16),
            grid=(N,),
            in_specs=[pl.BlockSpec((1, H + 2, W + 2, C), lambda i: (i, 0, 0, 0)),
                      wspec, sspec, sspec],
            out_specs=pl.BlockSpec((1, Ho, Wo, C), lambda i: (i, 0, 0, 0)),
            compiler_params=pltpu.CompilerParams(
                dimension_semantics=("parallel",)),
        )(xp, w, scale, bias)
    else:
        Ho = (H + 2 - 3) // 2 + 1
        Wo = (W + 2 - 3) // 2 + 1
        Hp, Wp = H + 2, W + 2
        xv = xp.reshape(N, Hp // 2, 2, Wp // 2, 2 * C)
        qspec0 = pl.BlockSpec((1, Hp // 2, 1, Wp // 2, 2 * C),
                              lambda i: (i, 0, 0, 0, 0))
        qspec1 = pl.BlockSpec((1, Hp // 2, 1, Wp // 2, 2 * C),
                              lambda i: (i, 0, 1, 0, 0))
        out = pl.pallas_call(
            functools.partial(_dw_s2_body, Ho=Ho, Wo=Wo, C=C),
            out_shape=jax.ShapeDtypeStruct((N, Ho, Wo, C), jnp.bfloat16),
            grid=(N,),
            in_specs=[qspec0, qspec1, wspec, sspec, sspec],
            out_specs=pl.BlockSpec((1, Ho, Wo, C), lambda i: (i, 0, 0, 0)),
            compiler_params=pltpu.CompilerParams(
                dimension_semantics=("parallel",)),
        )(xv, xv, w, scale, bias)
    return out


def _convd_body(m_ref, h_ref, w_ref, s_ref, c_ref, o_ref, *,
                k, TR, Wo, act):
    C = m_ref.shape[3]
    xw = jnp.concatenate([m_ref[0], h_ref[0]], axis=0)
    acc = None
    for dy in range(k):
        for dx in range(k):
            a = xw[dy:dy + TR, dx:dx + Wo, :].reshape(TR * Wo, C)
            t = (dy * k + dx) * C
            y = jnp.dot(a, w_ref[t:t + C, :],
                        preferred_element_type=jnp.float32)
            acc = y if acc is None else acc + y
    y = _act(acc * s_ref[...] + c_ref[...], act)
    o_ref[0] = y.reshape(TR, Wo, o_ref.shape[3]).astype(o_ref.dtype)


def _conv_direct(x, w, k, scale, bias, act, out_dtype=jnp.bfloat16):
    x = x.astype(jnp.bfloat16)
    N, H, W, C = x.shape
    Cout = w.shape[1]
    Ho, Wo = H - k + 1, W - k + 1
    Wop = _ru(Wo, 16)
    TR = min(32, _ru(Ho, 8))
    nt = -(-Ho // TR)
    xp = jnp.pad(x, ((0, 0), (0, TR * nt + 8 - H), (0, Wop + k - 1 - W),
                     (0, 0)))
    Lc = Wop + k - 1
    out = pl.pallas_call(
        functools.partial(_convd_body, k=k, TR=TR, Wo=Wop, act=act),
        out_shape=jax.ShapeDtypeStruct((N, TR * nt, Wop, Cout), out_dtype),
        grid=(N, nt),
        in_specs=[
            pl.BlockSpec((1, TR, Lc, C), lambda i, j: (i, j, 0, 0)),
            pl.BlockSpec((1, 8, Lc, C),
                         lambda i, j: (i, (j * TR + TR) // 8, 0, 0)),
            pl.BlockSpec((k * k * C, Cout), lambda i, j: (0, 0)),
            pl.BlockSpec((1, Cout), lambda i, j: (0, 0)),
            pl.BlockSpec((1, Cout), lambda i, j: (0, 0)),
        ],
        out_specs=pl.BlockSpec((1, TR, Wop, Cout), lambda i, j: (i, j, 0, 0)),
        compiler_params=pltpu.CompilerParams(
            dimension_semantics=("parallel", "parallel")),
    )(xp, xp, w, scale, bias)
    return out[:, :Ho, :Wo, :]


def _pool_body(x_ref, o_ref):
    m = jnp.mean(x_ref[...].astype(jnp.float32), axis=1)
    o_ref[...] = jnp.maximum(m, 0.0)


def _avgpool_relu(x):
    N, HW, C = x.shape
    return pl.pallas_call(
        _pool_body,
        out_shape=jax.ShapeDtypeStruct((N, C), jnp.float32),
    )(x)


def _reparam_body(mu_ref, lv_ref, eps_ref, o_ref):
    o_ref[...] = eps_ref[...] * jnp.exp(0.5 * lv_ref[...]) + mu_ref[...]


def _reparam(mu, log_var, eps):
    return pl.pallas_call(
        _reparam_body,
        out_shape=jax.ShapeDtypeStruct(mu.shape, jnp.float32),
    )(mu, log_var, eps)


def _cdiv(a, b):
    return -(-a // b)


def _convt_out(hin, k, s, op):
    return (hin - 1) * s + k + op


def _convt_need(win, k, s, hin):
    a, b = win
    d_last = (hin - 1) * s
    d_lo = max(a - (k - 1), 0)
    d_hi = min(b - 1, d_last)
    i_lo = max(0, min(_cdiv(d_lo, s), hin - 1))
    i_hi = max(i_lo, min(d_hi // s, hin - 1))
    return (i_lo, i_hi + 1)


def _convt_window(x, in_off, w, k, s, hin, win, owr, owc, scale, bias):
    N, xr, xc, C = x.shape
    ar, br = owr
    ac, bc = owc
    ir_lo, ir_hi = _convt_need(owr, k, s, hin)
    ic_lo, ic_hi = _convt_need(owc, k, s, win)
    xs = x[:, ir_lo - in_off[0]:ir_hi - in_off[0],
           ic_lo - in_off[1]:ic_hi - in_off[1], :].astype(jnp.bfloat16)
    nr, nc = ir_hi - ir_lo, ic_hi - ic_lo
    Lr = (br - ar) + k - 1
    Lc = (bc - ac) + k - 1
    sr = ir_lo * s + (k - 1) - ar
    sc = ic_lo * s + (k - 1) - ac
    buf = jnp.zeros((N, Lr, Lc, C), jnp.bfloat16)
    buf = buf.at[:, sr:sr + (nr - 1) * s + 1:s,
                 sc:sc + (nc - 1) * s + 1:s, :].set(xs)
    return _conv_direct(buf, w, k, scale, bias, "leaky")


def _encode(x_nhwc, stem, blocks, head, post, fc_mu, fc_var):
    stem_w, stem_s, stem_b = stem
    x = _conv(x_nhwc, stem_w, 3, 3, 2, 1, stem_s, stem_b, "relu6")
    for blk, (stride, has_exp, use_res) in zip(blocks, _BLOCK_CFG):
        inp = x
        h = x
        if has_exp:
            h = _conv(h, blk["exp_w"], 1, 1, 1, 0,
                      blk["exp_s"], blk["exp_b"], "relu6")
        h = _depthwise(h, blk["dw_w"], blk["dw_s"], blk["dw_b"], stride)
        x = _conv(h, blk["proj_w"], 1, 1, 1, 0,
                  blk["proj_s"], blk["proj_b"], "linear",
                  residual=inp if use_res else None)
    head_w, head_s, head_b = head
    x = _conv(x, head_w, 1, 1, 1, 0, head_s, head_b, "relu6")
    N, H, W, C = x.shape
    feat = _avgpool_relu(x.reshape(N, H * W, C))
    ones = jnp.ones((1, _LAT), jnp.float32)
    r = _mm_fused(feat.astype(jnp.bfloat16), post[0], ones, post[1],
                  "leaky", out_dtype=jnp.float32)
    mu = _mm_fused(r.astype(jnp.bfloat16), fc_mu[0], ones, fc_mu[1],
                   "linear", out_dtype=jnp.float32)
    log_var = _mm_fused(r.astype(jnp.bfloat16), fc_var[0], ones, fc_var[1],
                        "linear", out_dtype=jnp.float32)
    return mu, log_var


_DEC_CFG = [
    (_LAT // 4, 64, 5, 2, 0),
    (64, 64, 3, 2, 0),
    (64, 32, 3, 2, 0),
    (32, 16, 5, 3, 0),
    (16, 8, 3, 2, 0),
    (8, 8, 3, 2, 1),
]


def _decode(z, dec, final_w, final_b):
    N = z.shape[0]
    x = jnp.transpose(z.reshape(N, _LAT // 4, 2, 2), (0, 2, 3, 1))
    sizes = [(2, 2)]
    for (_, _, k, s, op) in _DEC_CFG:
        h, w = sizes[-1]
        sizes.append((_convt_out(h, k, s, op), _convt_out(w, k, s, op)))
    nly = len(_DEC_CFG)
    wins = [None] * nly
    wins[-1] = ((_CROP_R[0] - 1, _CROP_R[1] + 1),
                (_CROP_C[0] - 1, _CROP_C[1] + 1))
    for li in range(nly - 1, 0, -1):
        (_, _, k, s, _) = _DEC_CFG[li]
        hin, win = sizes[li]
        wins[li - 1] = (_convt_need(wins[li][0], k, s, hin),
                        _convt_need(wins[li][1], k, s, win))
    in_off = (0, 0)
    for li, ((_, _, k, s, _), ly) in enumerate(zip(_DEC_CFG, dec)):
        hin, win = sizes[li]
        owr, owc = wins[li]
        x = _convt_window(x, in_off, ly["w"], k, s, hin, win, owr, owc,
                          ly["scale"], ly["bias"])
        in_off = (owr[0], owc[0])
    ones = jnp.ones((1, _NCLS), jnp.float32)
    y = _conv_direct(x, final_w, 3, ones, final_b, "linear",
                     out_dtype=jnp.float32)
    return jnp.transpose(y, (0, 3, 1, 2))


def _unpack(params):
    it = iter(params[0:150])
    blocks = []
    for (stride, has_exp, use_res) in _BLOCK_CFG:
        blk = {}
        blk["dw_s"], blk["dw_b"], blk["dw_w"] = next(it), next(it), next(it)
        if has_exp:
            blk["exp_s"], blk["exp_b"], blk["exp_w"] = \
                next(it), next(it), next(it)
        blk["proj_s"], blk["proj_b"], blk["proj_w"] = \
            next(it), next(it), next(it)
        blocks.append(blk)
    dec = []
    for li in range(6):
        b, s, w = params[150 + 3 * li: 153 + 3 * li]
        dec.append({"bias": b, "scale": s, "w": w})
    (final_b, final_w, head_s, head_b, head_w, mu_b, mu_w,
     post_b, post_w, stem_s, stem_b, stem_w, var_b, var_w) = params[168:182]
    return dict(
        blocks=blocks, dec=dec, final_b=final_b, final_w=final_w,
        head=(head_w, head_s, head_b), fc_mu=(mu_w, mu_b),
        post=(post_w, post_b), stem=(stem_w, stem_s, stem_b),
        fc_var=(var_w, var_b))


def kernel(x, eps, *params):
    P = _unpack(params)
    x_nhwc = jnp.transpose(x, (0, 2, 3, 1)).astype(jnp.float32)
    mu, log_var = _encode(x_nhwc, P["stem"], P["blocks"], P["head"],
                          P["post"], P["fc_mu"], P["fc_var"])
    z = _reparam(mu, log_var, eps)
    dec_params = [dict(w=d["w"], scale=d["scale"], bias=d["bias"])
                  for d in P["dec"]]
    recons = _decode(z, dec_params, P["final_w"], P["final_b"])
    return recons, x, mu, log_var


# speedup vs baseline: 8.4799x; 1.0895x over previous
import functools

import jax
import jax.numpy as jnp
from jax.experimental import pallas as pl
from jax.experimental.pallas import tpu as pltpu

_LAT = 64
_NCLS = 3
_SLOPE = 0.01
_CROP_R = (150, 234)
_CROP_C = (24, 360)

_BLOCK_CFG = [
    (1, False, False),
    (2, True, False), (1, True, True),
    (2, True, False), (1, True, True), (1, True, True),
    (2, True, False), (1, True, True), (1, True, True), (1, True, True),
    (1, True, False), (1, True, True), (1, True, True),
    (2, True, False), (1, True, True), (1, True, True),
    (1, True, False),
]


def _ru(x, m):
    return (x + m - 1) // m * m


def _act(y, act):
    if act == "relu6":
        return jnp.clip(y, 0.0, 6.0)
    if act == "leaky":
        return jnp.where(y >= 0.0, y, _SLOPE * y)
    if act == "relu":
        return jnp.maximum(y, 0.0)
    return y


def _mm_body(*refs, act, has_res):
    if has_res:
        a_ref, b_ref, s_ref, c_ref, r_ref, o_ref = refs
    else:
        a_ref, b_ref, s_ref, c_ref, o_ref = refs
    y = jnp.dot(a_ref[...], b_ref[...], preferred_element_type=jnp.float32)
    y = _act(y * s_ref[...] + c_ref[...], act)
    if has_res:
        y = y + r_ref[...].astype(jnp.float32)
    o_ref[...] = y.astype(o_ref.dtype)


def _mm_fused(a, b, scale, bias, act="linear", residual=None,
              out_dtype=jnp.bfloat16):
    M, K = a.shape
    N = b.shape[1]
    has_res = residual is not None
    per_row = 2 * K + N * (4 if out_dtype == jnp.float32 else 2)
    if has_res:
        per_row += 2 * N
    bm = max(8, min(4096, (2 * 1024 * 1024 // per_row) // 8 * 8))
    bm = min(bm, _ru(M, 8))
    Mp = _ru(M, bm)
    if Mp != M:
        a = jnp.pad(a, ((0, Mp - M), (0, 0)))
        if has_res:
            residual = jnp.pad(residual, ((0, Mp - M), (0, 0)))
    specs = [
        pl.BlockSpec((bm, K), lambda i: (i, 0)),
        pl.BlockSpec((K, N), lambda i: (0, 0)),
        pl.BlockSpec((1, N), lambda i: (0, 0)),
        pl.BlockSpec((1, N), lambda i: (0, 0)),
    ]
    ops = [a, b, scale, bias]
    if has_res:
        specs.append(pl.BlockSpec((bm, N), lambda i: (i, 0)))
        ops.append(residual)
    out = pl.pallas_call(
        functools.partial(_mm_body, act=act, has_res=has_res),
        out_shape=jax.ShapeDtypeStruct((Mp, N), out_dtype),
        grid=(Mp // bm,),
        in_specs=specs,
        out_specs=pl.BlockSpec((bm, N), lambda i: (i, 0)),
        compiler_params=pltpu.CompilerParams(
            dimension_semantics=("parallel",)),
    )(*ops)
    return out[:M] if Mp != M else out


def _im2col(x, kh, kw, stride, padding):
    if padding:
        x = jnp.pad(x, ((0, 0), (padding, padding), (padding, padding), (0, 0)))
    N, H, W, C = x.shape
    Ho = (H - kh) // stride + 1
    Wo = (W - kw) // stride + 1
    cols = [x[:, dy:dy + stride * Ho:stride, dx:dx + stride * Wo:stride, :]
            for dy in range(kh) for dx in range(kw)]
    patches = jnp.stack(cols, axis=3)
    return patches.reshape(N * Ho * Wo, kh * kw * C), (N, Ho, Wo)


def _conv(x, w, kh, kw, stride, padding, scale, bias, act,
          residual=None, out_dtype=jnp.bfloat16):
    x = x.astype(jnp.bfloat16)
    N, H, W, C = x.shape
    Cout = w.shape[1]
    if kh == 1 and kw == 1 and stride == 1 and padding == 0:
        patches, (N, Ho, Wo) = x.reshape(N * H * W, C), (N, H, W)
    else:
        patches, (N, Ho, Wo) = _im2col(x, kh, kw, stride, padding)
    res = residual.reshape(N * Ho * Wo, Cout) if residual is not None else None
    y = _mm_fused(patches, w, scale, bias, act, residual=res,
                  out_dtype=out_dtype)
    return y.reshape(N, Ho, Wo, Cout)


def _dw_s1_body(x_ref, w_ref, s_ref, c_ref, o_ref, *, Ho, Wo):
    xp = x_ref[0]
    acc = None
    for dy in range(3):
        for dx in range(3):
            t = xp[dy:dy + Ho, dx:dx + Wo, :].astype(jnp.float32)
            term = t * w_ref[3 * dy + dx]
            acc = term if acc is None else acc + term
    y = jnp.clip(acc * s_ref[...] + c_ref[...], 0.0, 6.0)
    o_ref[0] = y.astype(jnp.bfloat16)


def _dw_s2_body(q0_ref, q1_ref, w_ref, s_ref, c_ref, o_ref, *, Ho, Wo, C):
    pb = (q0_ref[0, :, 0], q1_ref[0, :, 0])
    acc = None
    for dy in range(3):
        for dx in range(3):
            r = dx % 2
            ph = pb[dy % 2][:, :, r * C:(r + 1) * C]
            t = ph[dy // 2:dy // 2 + Ho, dx // 2:dx // 2 + Wo, :]
            term = t.astype(jnp.float32) * w_ref[3 * dy + dx]
            acc = term if acc is None else acc + term
    y = jnp.clip(acc * s_ref[...] + c_ref[...], 0.0, 6.0)
    o_ref[0] = y.astype(jnp.bfloat16)


def _depthwise(x, w9, scale, bias, stride):
    N, H, W, C = x.shape
    w = w9.reshape(9, C)
    xp = jnp.pad(x, ((0, 0), (1, 1), (1, 1), (0, 0)))
    wspec = pl.BlockSpec((9, C), lambda i: (0, 0))
    sspec = pl.BlockSpec((1, C), lambda i: (0, 0))
    if stride == 1:
        Ho, Wo = H, W
        out = pl.pallas_call(
            functools.partial(_dw_s1_body, Ho=Ho, Wo=Wo),
            out_shape=jax.ShapeDtypeStruct((N, Ho, Wo, C), jnp.bfloat16),
            grid=(N,),
            in_specs=[pl.BlockSpec((1, H + 2, W + 2, C), lambda i: (i, 0, 0, 0)),
                      wspec, sspec, sspec],
            out_specs=pl.BlockSpec((1, Ho, Wo, C), lambda i: (i, 0, 0, 0)),
            compiler_params=pltpu.CompilerParams(
                dimension_semantics=("parallel",)),
        )(xp, w, scale, bias)
    else:
        Ho = (H + 2 - 3) // 2 + 1
        Wo = (W + 2 - 3) // 2 + 1
        Hp, Wp = H + 2, W + 2
        xv = xp.reshape(N, Hp // 2, 2, Wp // 2, 2 * C)
        qspec0 = pl.BlockSpec((1, Hp // 2, 1, Wp // 2, 2 * C),
                              lambda i: (i, 0, 0, 0, 0))
        qspec1 = pl.BlockSpec((1, Hp // 2, 1, Wp // 2, 2 * C),
                              lambda i: (i, 0, 1, 0, 0))
        out = pl.pallas_call(
            functools.partial(_dw_s2_body, Ho=Ho, Wo=Wo, C=C),
            out_shape=jax.ShapeDtypeStruct((N, Ho, Wo, C), jnp.bfloat16),
            grid=(N,),
            in_specs=[qspec0, qspec1, wspec, sspec, sspec],
            out_specs=pl.BlockSpec((1, Ho, Wo, C), lambda i: (i, 0, 0, 0)),
            compiler_params=pltpu.CompilerParams(
                dimension_semantics=("parallel",)),
        )(xv, xv, w, scale, bias)
    return out


def _stem_body(q0_ref, q1_ref, w_ref, s_ref, c_ref, o_ref, *, Ho, Wo, C):
    pb = (q0_ref[0, :, 0], q1_ref[0, :, 0])
    taps = []
    for dy in range(3):
        for dx in range(3):
            r = dx % 2
            ph = pb[dy % 2][:, :, r * C:(r + 1) * C]
            taps.append(ph[dy // 2:dy // 2 + Ho, dx // 2:dx // 2 + Wo, :])
    a = jnp.concatenate(taps, axis=-1).reshape(Ho * Wo, 9 * C)
    y = jnp.dot(a, w_ref[...], preferred_element_type=jnp.float32)
    y = jnp.clip(y * s_ref[...] + c_ref[...], 0.0, 6.0)
    o_ref[0] = y.reshape(Ho, Wo, o_ref.shape[3]).astype(jnp.bfloat16)


def _stem_conv(x, w, scale, bias):
    N, H, W, C = x.shape
    Cout = w.shape[1]
    Ho, Wo = H // 2, W // 2
    xp = jnp.pad(x.astype(jnp.bfloat16), ((0, 0), (1, 1), (1, 1), (0, 0)))
    Hp, Wp = H + 2, W + 2
    xv = xp.reshape(N, Hp // 2, 2, Wp // 2, 2 * C)
    qspec0 = pl.BlockSpec((1, Hp // 2, 1, Wp // 2, 2 * C),
                          lambda i: (i, 0, 0, 0, 0))
    qspec1 = pl.BlockSpec((1, Hp // 2, 1, Wp // 2, 2 * C),
                          lambda i: (i, 0, 1, 0, 0))
    return pl.pallas_call(
        functools.partial(_stem_body, Ho=Ho, Wo=Wo, C=C),
        out_shape=jax.ShapeDtypeStruct((N, Ho, Wo, Cout), jnp.bfloat16),
        grid=(N,),
        in_specs=[qspec0, qspec1,
                  pl.BlockSpec((9 * C, Cout), lambda i: (0, 0)),
                  pl.BlockSpec((1, Cout), lambda i: (0, 0)),
                  pl.BlockSpec((1, Cout), lambda i: (0, 0))],
        out_specs=pl.BlockSpec((1, Ho, Wo, Cout), lambda i: (i, 0, 0, 0)),
        compiler_params=pltpu.CompilerParams(
            dimension_semantics=("parallel",)),
    )(xv, xv, w, scale, bias)


def _convd_body(m_ref, h_ref, w_ref, s_ref, c_ref, o_ref, *,
                k, TR, Wo, act):
    C = m_ref.shape[3]
    xw = jnp.concatenate([m_ref[0], h_ref[0]], axis=0)
    acc = None
    for dy in range(k):
        for dx in range(k):
            a = xw[dy:dy + TR, dx:dx + Wo, :].reshape(TR * Wo, C)
            t = (dy * k + dx) * C
            y = jnp.dot(a, w_ref[t:t + C, :],
                        preferred_element_type=jnp.float32)
            acc = y if acc is None else acc + y
    y = _act(acc * s_ref[...] + c_ref[...], act)
    o_ref[0] = y.reshape(TR, Wo, o_ref.shape[3]).astype(o_ref.dtype)


def _conv_direct(x, w, k, scale, bias, act, out_dtype=jnp.bfloat16):
    x = x.astype(jnp.bfloat16)
    N, H, W, C = x.shape
    Cout = w.shape[1]
    Ho, Wo = H - k + 1, W - k + 1
    Wop = _ru(Wo, 16)
    TR = min(32, _ru(Ho, 8))
    nt = -(-Ho // TR)
    xp = jnp.pad(x, ((0, 0), (0, TR * nt + 8 - H), (0, Wop + k - 1 - W),
                     (0, 0)))
    Lc = Wop + k - 1
    out = pl.pallas_call(
        functools.partial(_convd_body, k=k, TR=TR, Wo=Wop, act=act),
        out_shape=jax.ShapeDtypeStruct((N, TR * nt, Wop, Cout), out_dtype),
        grid=(N, nt),
        in_specs=[
            pl.BlockSpec((1, TR, Lc, C), lambda i, j: (i, j, 0, 0)),
            pl.BlockSpec((1, 8, Lc, C),
                         lambda i, j: (i, (j * TR + TR) // 8, 0, 0)),
            pl.BlockSpec((k * k * C, Cout), lambda i, j: (0, 0)),
            pl.BlockSpec((1, Cout), lambda i, j: (0, 0)),
            pl.BlockSpec((1, Cout), lambda i, j: (0, 0)),
        ],
        out_specs=pl.BlockSpec((1, TR, Wop, Cout), lambda i, j: (i, j, 0, 0)),
        compiler_params=pltpu.CompilerParams(
            dimension_semantics=("parallel", "parallel")),
    )(xp, xp, w, scale, bias)
    return out[:, :Ho, :Wo, :]


def _pool_body(x_ref, o_ref):
    m = jnp.mean(x_ref[...].astype(jnp.float32), axis=1)
    o_ref[...] = jnp.maximum(m, 0.0)


def _avgpool_relu(x):
    N, HW, C = x.shape
    return pl.pallas_call(
        _pool_body,
        out_shape=jax.ShapeDtypeStruct((N, C), jnp.float32),
    )(x)


def _reparam_body(mu_ref, lv_ref, eps_ref, o_ref):
    o_ref[...] = eps_ref[...] * jnp.exp(0.5 * lv_ref[...]) + mu_ref[...]


def _reparam(mu, log_var, eps):
    return pl.pallas_call(
        _reparam_body,
        out_shape=jax.ShapeDtypeStruct(mu.shape, jnp.float32),
    )(mu, log_var, eps)


def _cdiv(a, b):
    return -(-a // b)


def _convt_out(hin, k, s, op):
    return (hin - 1) * s + k + op


def _convt_need(win, k, s, hin):
    a, b = win
    d_last = (hin - 1) * s
    d_lo = max(a - (k - 1), 0)
    d_hi = min(b - 1, d_last)
    i_lo = max(0, min(_cdiv(d_lo, s), hin - 1))
    i_hi = max(i_lo, min(d_hi // s, hin - 1))
    return (i_lo, i_hi + 1)


def _convt_window(x, in_off, w, k, s, hin, win, owr, owc, scale, bias):
    N, xr, xc, C = x.shape
    ar, br = owr
    ac, bc = owc
    ir_lo, ir_hi = _convt_need(owr, k, s, hin)
    ic_lo, ic_hi = _convt_need(owc, k, s, win)
    xs = x[:, ir_lo - in_off[0]:ir_hi - in_off[0],
           ic_lo - in_off[1]:ic_hi - in_off[1], :].astype(jnp.bfloat16)
    nr, nc = ir_hi - ir_lo, ic_hi - ic_lo
    Lr = (br - ar) + k - 1
    Lc = (bc - ac) + k - 1
    sr = ir_lo * s + (k - 1) - ar
    sc = ic_lo * s + (k - 1) - ac
    buf = jnp.zeros((N, Lr, Lc, C), jnp.bfloat16)
    buf = buf.at[:, sr:sr + (nr - 1) * s + 1:s,
                 sc:sc + (nc - 1) * s + 1:s, :].set(xs)
    return _conv_direct(buf, w, k, scale, bias, "leaky")


def _encode(x_nhwc, stem, blocks, head, post, fc_mu, fc_var):
    stem_w, stem_s, stem_b = stem
    x = _stem_conv(x_nhwc, stem_w, stem_s, stem_b)
    for blk, (stride, has_exp, use_res) in zip(blocks, _BLOCK_CFG):
        inp = x
        h = x
        if has_exp:
            h = _conv(h, blk["exp_w"], 1, 1, 1, 0,
                      blk["exp_s"], blk["exp_b"], "relu6")
        h = _depthwise(h, blk["dw_w"], blk["dw_s"], blk["dw_b"], stride)
        x = _conv(h, blk["proj_w"], 1, 1, 1, 0,
                  blk["proj_s"], blk["proj_b"], "linear",
                  residual=inp if use_res else None)
    head_w, head_s, head_b = head
    x = _conv(x, head_w, 1, 1, 1, 0, head_s, head_b, "relu6")
    N, H, W, C = x.shape
    feat = _avgpool_relu(x.reshape(N, H * W, C))
    ones = jnp.ones((1, _LAT), jnp.float32)
    r = _mm_fused(feat.astype(jnp.bfloat16), post[0], ones, post[1],
                  "leaky", out_dtype=jnp.float32)
    mu = _mm_fused(r.astype(jnp.bfloat16), fc_mu[0], ones, fc_mu[1],
                   "linear", out_dtype=jnp.float32)
    log_var = _mm_fused(r.astype(jnp.bfloat16), fc_var[0], ones, fc_var[1],
                        "linear", out_dtype=jnp.float32)
    return mu, log_var


_DEC_CFG = [
    (_LAT // 4, 64, 5, 2, 0),
    (64, 64, 3, 2, 0),
    (64, 32, 3, 2, 0),
    (32, 16, 5, 3, 0),
    (16, 8, 3, 2, 0),
    (8, 8, 3, 2, 1),
]


def _decode(z, dec, final_w, final_b):
    N = z.shape[0]
    x = jnp.transpose(z.reshape(N, _LAT // 4, 2, 2), (0, 2, 3, 1))
    sizes = [(2, 2)]
    for (_, _, k, s, op) in _DEC_CFG:
        h, w = sizes[-1]
        sizes.append((_convt_out(h, k, s, op), _convt_out(w, k, s, op)))
    nly = len(_DEC_CFG)
    wins = [None] * nly
    wins[-1] = ((_CROP_R[0] - 1, _CROP_R[1] + 1),
                (_CROP_C[0] - 1, _CROP_C[1] + 1))
    for li in range(nly - 1, 0, -1):
        (_, _, k, s, _) = _DEC_CFG[li]
        hin, win = sizes[li]
        wins[li - 1] = (_convt_need(wins[li][0], k, s, hin),
                        _convt_need(wins[li][1], k, s, win))
    in_off = (0, 0)
    for li, ((_, _, k, s, _), ly) in enumerate(zip(_DEC_CFG, dec)):
        hin, win = sizes[li]
        owr, owc = wins[li]
        x = _convt_window(x, in_off, ly["w"], k, s, hin, win, owr, owc,
                          ly["scale"], ly["bias"])
        in_off = (owr[0], owc[0])
    ones = jnp.ones((1, _NCLS), jnp.float32)
    y = _conv_direct(x, final_w, 3, ones, final_b, "linear",
                     out_dtype=jnp.float32)
    return jnp.transpose(y, (0, 3, 1, 2))


def _unpack(params):
    it = iter(params[0:150])
    blocks = []
    for (stride, has_exp, use_res) in _BLOCK_CFG:
        blk = {}
        blk["dw_s"], blk["dw_b"], blk["dw_w"] = next(it), next(it), next(it)
        if has_exp:
            blk["exp_s"], blk["exp_b"], blk["exp_w"] = \
                next(it), next(it), next(it)
        blk["proj_s"], blk["proj_b"], blk["proj_w"] = \
            next(it), next(it), next(it)
        blocks.append(blk)
    dec = []
    for li in range(6):
        b, s, w = params[150 + 3 * li: 153 + 3 * li]
        dec.append({"bias": b, "scale": s, "w": w})
    (final_b, final_w, head_s, head_b, head_w, mu_b, mu_w,
     post_b, post_w, stem_s, stem_b, stem_w, var_b, var_w) = params[168:182]
    return dict(
        blocks=blocks, dec=dec, final_b=final_b, final_w=final_w,
        head=(head_w, head_s, head_b), fc_mu=(mu_w, mu_b),
        post=(post_w, post_b), stem=(stem_w, stem_s, stem_b),
        fc_var=(var_w, var_b))


def kernel(x, eps, *params):
    P = _unpack(params)
    x_nhwc = jnp.transpose(x, (0, 2, 3, 1)).astype(jnp.float32)
    mu, log_var = _encode(x_nhwc, P["stem"], P["blocks"], P["head"],
                          P["post"], P["fc_mu"], P["fc_var"])
    z = _reparam(mu, log_var, eps)
    dec_params = [dict(w=d["w"], scale=d["scale"], bias=d["bias"])
                  for d in P["dec"]]
    recons = _decode(z, dec_params, P["final_w"], P["final_b"])
    return recons, x, mu, log_var


# speedup vs baseline: 9.4440x; 1.1137x over previous
import functools

import jax
import jax.numpy as jnp
from jax.experimental import pallas as pl
from jax.experimental.pallas import tpu as pltpu

_LAT = 64
_NCLS = 3
_SLOPE = 0.01
_CROP_R = (150, 234)
_CROP_C = (24, 360)

_BLOCK_CFG = [
    (1, False, False),
    (2, True, False), (1, True, True),
    (2, True, False), (1, True, True), (1, True, True),
    (2, True, False), (1, True, True), (1, True, True), (1, True, True),
    (1, True, False), (1, True, True), (1, True, True),
    (2, True, False), (1, True, True), (1, True, True),
    (1, True, False),
]


def _ru(x, m):
    return (x + m - 1) // m * m


def _act(y, act):
    if act == "relu6":
        return jnp.clip(y, 0.0, 6.0)
    if act == "leaky":
        return jnp.where(y >= 0.0, y, _SLOPE * y)
    if act == "relu":
        return jnp.maximum(y, 0.0)
    return y


def _mm_body(*refs, act, has_res):
    if has_res:
        a_ref, b_ref, s_ref, c_ref, r_ref, o_ref = refs
    else:
        a_ref, b_ref, s_ref, c_ref, o_ref = refs
    y = jnp.dot(a_ref[...], b_ref[...], preferred_element_type=jnp.float32)
    y = _act(y * s_ref[...] + c_ref[...], act)
    if has_res:
        y = y + r_ref[...].astype(jnp.float32)
    o_ref[...] = y.astype(o_ref.dtype)


def _mm_fused(a, b, scale, bias, act="linear", residual=None,
              out_dtype=jnp.bfloat16):
    M, K = a.shape
    N = b.shape[1]
    has_res = residual is not None
    per_row = 2 * K + N * (4 if out_dtype == jnp.float32 else 2)
    if has_res:
        per_row += 2 * N
    bm = max(8, min(4096, (2 * 1024 * 1024 // per_row) // 8 * 8))
    bm = min(bm, _ru(M, 8))
    Mp = _ru(M, bm)
    if Mp != M:
        a = jnp.pad(a, ((0, Mp - M), (0, 0)))
        if has_res:
            residual = jnp.pad(residual, ((0, Mp - M), (0, 0)))
    specs = [
        pl.BlockSpec((bm, K), lambda i: (i, 0)),
        pl.BlockSpec((K, N), lambda i: (0, 0)),
        pl.BlockSpec((1, N), lambda i: (0, 0)),
        pl.BlockSpec((1, N), lambda i: (0, 0)),
    ]
    ops = [a, b, scale, bias]
    if has_res:
        specs.append(pl.BlockSpec((bm, N), lambda i: (i, 0)))
        ops.append(residual)
    out = pl.pallas_call(
        functools.partial(_mm_body, act=act, has_res=has_res),
        out_shape=jax.ShapeDtypeStruct((Mp, N), out_dtype),
        grid=(Mp // bm,),
        in_specs=specs,
        out_specs=pl.BlockSpec((bm, N), lambda i: (i, 0)),
        compiler_params=pltpu.CompilerParams(
            dimension_semantics=("parallel",)),
    )(*ops)
    return out[:M] if Mp != M else out


def _im2col(x, kh, kw, stride, padding):
    if padding:
        x = jnp.pad(x, ((0, 0), (padding, padding), (padding, padding), (0, 0)))
    N, H, W, C = x.shape
    Ho = (H - kh) // stride + 1
    Wo = (W - kw) // stride + 1
    cols = [x[:, dy:dy + stride * Ho:stride, dx:dx + stride * Wo:stride, :]
            for dy in range(kh) for dx in range(kw)]
    patches = jnp.stack(cols, axis=3)
    return patches.reshape(N * Ho * Wo, kh * kw * C), (N, Ho, Wo)


def _conv(x, w, kh, kw, stride, padding, scale, bias, act,
          residual=None, out_dtype=jnp.bfloat16):
    x = x.astype(jnp.bfloat16)
    N, H, W, C = x.shape
    Cout = w.shape[1]
    if kh == 1 and kw == 1 and stride == 1 and padding == 0:
        patches, (N, Ho, Wo) = x.reshape(N * H * W, C), (N, H, W)
    else:
        patches, (N, Ho, Wo) = _im2col(x, kh, kw, stride, padding)
    res = residual.reshape(N * Ho * Wo, Cout) if residual is not None else None
    y = _mm_fused(patches, w, scale, bias, act, residual=res,
                  out_dtype=out_dtype)
    return y.reshape(N, Ho, Wo, Cout)


def _dw_s1_body(x_ref, w_ref, s_ref, c_ref, o_ref, *, Ho, Wo):
    xp = x_ref[0]
    acc = None
    for dy in range(3):
        for dx in range(3):
            t = xp[dy:dy + Ho, dx:dx + Wo, :].astype(jnp.float32)
            term = t * w_ref[3 * dy + dx]
            acc = term if acc is None else acc + term
    y = jnp.clip(acc * s_ref[...] + c_ref[...], 0.0, 6.0)
    o_ref[0] = y.astype(jnp.bfloat16)


def _dw_s2_body(q0_ref, q1_ref, w_ref, s_ref, c_ref, o_ref, *, Ho, Wo, C):
    pb = (q0_ref[0, :, 0], q1_ref[0, :, 0])
    acc = None
    for dy in range(3):
        for dx in range(3):
            r = dx % 2
            ph = pb[dy % 2][:, :, r * C:(r + 1) * C]
            t = ph[dy // 2:dy // 2 + Ho, dx // 2:dx // 2 + Wo, :]
            term = t.astype(jnp.float32) * w_ref[3 * dy + dx]
            acc = term if acc is None else acc + term
    y = jnp.clip(acc * s_ref[...] + c_ref[...], 0.0, 6.0)
    o_ref[0] = y.astype(jnp.bfloat16)


def _depthwise(x, w9, scale, bias, stride):
    N, H, W, C = x.shape
    w = w9.reshape(9, C)
    xp = jnp.pad(x, ((0, 0), (1, 1), (1, 1), (0, 0)))
    wspec = pl.BlockSpec((9, C), lambda i: (0, 0))
    sspec = pl.BlockSpec((1, C), lambda i: (0, 0))
    if stride == 1:
        Ho, Wo = H, W
        out = pl.pallas_call(
            functools.partial(_dw_s1_body, Ho=Ho, Wo=Wo),
            out_shape=jax.ShapeDtypeStruct((N, Ho, Wo, C), jnp.bfloat16),
            grid=(N,),
            in_specs=[pl.BlockSpec((1, H + 2, W + 2, C), lambda i: (i, 0, 0, 0)),
                      wspec, sspec, sspec],
            out_specs=pl.BlockSpec((1, Ho, Wo, C), lambda i: (i, 0, 0, 0)),
            compiler_params=pltpu.CompilerParams(
                dimension_semantics=("parallel",)),
        )(xp, w, scale, bias)
    else:
        Ho = (H + 2 - 3) // 2 + 1
        Wo = (W + 2 - 3) // 2 + 1
        Hp, Wp = H + 2, W + 2
        xv = xp.reshape(N, Hp // 2, 2, Wp // 2, 2 * C)
        qspec0 = pl.BlockSpec((1, Hp // 2, 1, Wp // 2, 2 * C),
                              lambda i: (i, 0, 0, 0, 0))
        qspec1 = pl.BlockSpec((1, Hp // 2, 1, Wp // 2, 2 * C),
                              lambda i: (i, 0, 1, 0, 0))
        out = pl.pallas_call(
            functools.partial(_dw_s2_body, Ho=Ho, Wo=Wo, C=C),
            out_shape=jax.ShapeDtypeStruct((N, Ho, Wo, C), jnp.bfloat16),
            grid=(N,),
            in_specs=[qspec0, qspec1, wspec, sspec, sspec],
            out_specs=pl.BlockSpec((1, Ho, Wo, C), lambda i: (i, 0, 0, 0)),
            compiler_params=pltpu.CompilerParams(
                dimension_semantics=("parallel",)),
        )(xv, xv, w, scale, bias)
    return out


def _stem_body(q0_ref, q1_ref, w_ref, s_ref, c_ref, o_ref, *, Ho, Wo, C):
    pb = (q0_ref[0, :, 0], q1_ref[0, :, 0])
    taps = []
    for dy in range(3):
        for dx in range(3):
            r = dx % 2
            ph = pb[dy % 2][:, :, r * C:(r + 1) * C]
            taps.append(ph[dy // 2:dy // 2 + Ho, dx // 2:dx // 2 + Wo, :])
    a = jnp.concatenate(taps, axis=-1).reshape(Ho * Wo, 9 * C)
    y = jnp.dot(a, w_ref[...], preferred_element_type=jnp.float32)
    y = jnp.clip(y * s_ref[...] + c_ref[...], 0.0, 6.0)
    o_ref[0] = y.reshape(Ho, Wo, o_ref.shape[3]).astype(jnp.bfloat16)


def _stem_conv(x, w, scale, bias):
    N, H, W, C = x.shape
    Cout = w.shape[1]
    Ho, Wo = H // 2, W // 2
    xp = jnp.pad(x.astype(jnp.bfloat16), ((0, 0), (1, 1), (1, 1), (0, 0)))
    Hp, Wp = H + 2, W + 2
    xv = xp.reshape(N, Hp // 2, 2, Wp // 2, 2 * C)
    qspec0 = pl.BlockSpec((1, Hp // 2, 1, Wp // 2, 2 * C),
                          lambda i: (i, 0, 0, 0, 0))
    qspec1 = pl.BlockSpec((1, Hp // 2, 1, Wp // 2, 2 * C),
                          lambda i: (i, 0, 1, 0, 0))
    return pl.pallas_call(
        functools.partial(_stem_body, Ho=Ho, Wo=Wo, C=C),
        out_shape=jax.ShapeDtypeStruct((N, Ho, Wo, Cout), jnp.bfloat16),
        grid=(N,),
        in_specs=[qspec0, qspec1,
                  pl.BlockSpec((9 * C, Cout), lambda i: (0, 0)),
                  pl.BlockSpec((1, Cout), lambda i: (0, 0)),
                  pl.BlockSpec((1, Cout), lambda i: (0, 0))],
        out_specs=pl.BlockSpec((1, Ho, Wo, Cout), lambda i: (i, 0, 0, 0)),
        compiler_params=pltpu.CompilerParams(
            dimension_semantics=("parallel",)),
    )(xv, xv, w, scale, bias)


def _convd_body(m_ref, h_ref, w_ref, s_ref, c_ref, o_ref, *,
                k, TR, Wo, act):
    C = m_ref.shape[3]
    xw = jnp.concatenate([m_ref[0], h_ref[0]], axis=0)
    taps = [xw[dy:dy + TR, dx:dx + Wo, :]
            for dy in range(k) for dx in range(k)]
    a = jnp.concatenate(taps, axis=-1).reshape(TR * Wo, k * k * C)
    y = jnp.dot(a, w_ref[...], preferred_element_type=jnp.float32)
    y = _act(y * s_ref[...] + c_ref[...], act)
    o_ref[0] = y.reshape(TR, Wo, o_ref.shape[3]).astype(o_ref.dtype)


def _conv_direct(x, w, k, scale, bias, act, out_dtype=jnp.bfloat16,
                 out_hw=None):
    x = x.astype(jnp.bfloat16)
    N, H, W, C = x.shape
    Cout = w.shape[1]
    Ho, Wo = out_hw if out_hw is not None else (H - k + 1, W - k + 1)
    Wop = _ru(Wo, 16)
    TR = min(32, _ru(Ho, 8))
    nt = -(-Ho // TR)
    xp = jnp.pad(x, ((0, 0), (0, max(0, TR * nt + 8 - H)),
                     (0, max(0, Wop + k - 1 - W)), (0, 0)))
    Lc = xp.shape[2]
    out = pl.pallas_call(
        functools.partial(_convd_body, k=k, TR=TR, Wo=Wop, act=act),
        out_shape=jax.ShapeDtypeStruct((N, TR * nt, Wop, Cout), out_dtype),
        grid=(N, nt),
        in_specs=[
            pl.BlockSpec((1, TR, Lc, C), lambda i, j: (i, j, 0, 0)),
            pl.BlockSpec((1, 8, Lc, C),
                         lambda i, j: (i, (j * TR + TR) // 8, 0, 0)),
            pl.BlockSpec((k * k * C, Cout), lambda i, j: (0, 0)),
            pl.BlockSpec((1, Cout), lambda i, j: (0, 0)),
            pl.BlockSpec((1, Cout), lambda i, j: (0, 0)),
        ],
        out_specs=pl.BlockSpec((1, TR, Wop, Cout), lambda i, j: (i, j, 0, 0)),
        compiler_params=pltpu.CompilerParams(
            dimension_semantics=("parallel", "parallel")),
    )(xp, xp, w, scale, bias)
    return out[:, :Ho, :Wo, :]


def _pool_body(x_ref, o_ref):
    m = jnp.mean(x_ref[...].astype(jnp.float32), axis=1)
    o_ref[...] = jnp.maximum(m, 0.0)


def _avgpool_relu(x):
    N, HW, C = x.shape
    return pl.pallas_call(
        _pool_body,
        out_shape=jax.ShapeDtypeStruct((N, C), jnp.float32),
    )(x)


def _reparam_body(mu_ref, lv_ref, eps_ref, o_ref):
    o_ref[...] = eps_ref[...] * jnp.exp(0.5 * lv_ref[...]) + mu_ref[...]


def _reparam(mu, log_var, eps):
    return pl.pallas_call(
        _reparam_body,
        out_shape=jax.ShapeDtypeStruct(mu.shape, jnp.float32),
    )(mu, log_var, eps)


def _cdiv(a, b):
    return -(-a // b)


def _convt_out(hin, k, s, op):
    return (hin - 1) * s + k + op


def _convt_need(win, k, s, hin):
    a, b = win
    d_last = (hin - 1) * s
    d_lo = max(a - (k - 1), 0)
    d_hi = min(b - 1, d_last)
    i_lo = max(0, min(_cdiv(d_lo, s), hin - 1))
    i_hi = max(i_lo, min(d_hi // s, hin - 1))
    return (i_lo, i_hi + 1)


def _convt_window(x, in_off, w, k, s, hin, win, owr, owc, scale, bias):
    N, xr, xc, C = x.shape
    ar, br = owr
    ac, bc = owc
    ir_lo, ir_hi = _convt_need(owr, k, s, hin)
    ic_lo, ic_hi = _convt_need(owc, k, s, win)
    xs = x[:, ir_lo - in_off[0]:ir_hi - in_off[0],
           ic_lo - in_off[1]:ic_hi - in_off[1], :].astype(jnp.bfloat16)
    nr, nc = ir_hi - ir_lo, ic_hi - ic_lo
    Lr = (br - ar) + k - 1
    Lc = (bc - ac) + k - 1
    sr = ir_lo * s + (k - 1) - ar
    sc = ic_lo * s + (k - 1) - ac
    buf = _dilate(xs, s, (sr, sc), (Lr, Lc))
    return _conv_direct(buf, w, k, scale, bias, "leaky",
                        out_hw=(br - ar, bc - ac))


def _dil_body(x_ref, o_ref, *, qs, r, C, s):
    q = pl.program_id(1)
    xb = x_ref[0]
    pads = [jnp.zeros(xb.shape[:2] + (r * C,), jnp.bfloat16),
            xb,
            jnp.zeros(xb.shape[:2] + ((s - 1 - r) * C,), jnp.bfloat16)]
    y = jnp.concatenate([p for p in pads if p.shape[-1]], axis=-1)
    o_ref[0, :, 0] = jnp.where(q == qs, y, jnp.zeros_like(y))


def _dilate(xs, s, off, size):
    N, nr, nc, C = xs.shape
    Lr, Lc = _ru(size[0], s), _ru(size[1], s)
    sr, sc = off
    xsp = jnp.pad(xs, ((0, 0),
                       (sr // s, Lr // s - sr // s - nr),
                       (sc // s, Lc // s - sc // s - nc),
                       (0, 0)))
    out = pl.pallas_call(
        functools.partial(_dil_body, qs=sr % s, r=sc % s, C=C, s=s),
        out_shape=jax.ShapeDtypeStruct((N, Lr // s, s, Lc // s, s * C),
                                       jnp.bfloat16),
        grid=(N, s),
        in_specs=[pl.BlockSpec((1, Lr // s, Lc // s, C),
                               lambda i, q: (i, 0, 0, 0))],
        out_specs=pl.BlockSpec((1, Lr // s, 1, Lc // s, s * C),
                               lambda i, q: (i, 0, q, 0, 0)),
        compiler_params=pltpu.CompilerParams(
            dimension_semantics=("parallel", "parallel")),
    )(xsp)
    return out.reshape(N, Lr, Lc, C)


def _encode(x_nhwc, stem, blocks, head, post, fc_mu, fc_var):
    stem_w, stem_s, stem_b = stem
    x = _stem_conv(x_nhwc, stem_w, stem_s, stem_b)
    for blk, (stride, has_exp, use_res) in zip(blocks, _BLOCK_CFG):
        inp = x
        h = x
        if has_exp:
            h = _conv(h, blk["exp_w"], 1, 1, 1, 0,
                      blk["exp_s"], blk["exp_b"], "relu6")
        h = _depthwise(h, blk["dw_w"], blk["dw_s"], blk["dw_b"], stride)
        x = _conv(h, blk["proj_w"], 1, 1, 1, 0,
                  blk["proj_s"], blk["proj_b"], "linear",
                  residual=inp if use_res else None)
    head_w, head_s, head_b = head
    x = _conv(x, head_w, 1, 1, 1, 0, head_s, head_b, "relu6")
    N, H, W, C = x.shape
    feat = _avgpool_relu(x.reshape(N, H * W, C))
    ones = jnp.ones((1, _LAT), jnp.float32)
    r = _mm_fused(feat.astype(jnp.bfloat16), post[0], ones, post[1],
                  "leaky", out_dtype=jnp.float32)
    mu = _mm_fused(r.astype(jnp.bfloat16), fc_mu[0], ones, fc_mu[1],
                   "linear", out_dtype=jnp.float32)
    log_var = _mm_fused(r.astype(jnp.bfloat16), fc_var[0], ones, fc_var[1],
                        "linear", out_dtype=jnp.float32)
    return mu, log_var


_DEC_CFG = [
    (_LAT // 4, 64, 5, 2, 0),
    (64, 64, 3, 2, 0),
    (64, 32, 3, 2, 0),
    (32, 16, 5, 3, 0),
    (16, 8, 3, 2, 0),
    (8, 8, 3, 2, 1),
]


def _decode(z, dec, final_w, final_b):
    N = z.shape[0]
    x = jnp.transpose(z.reshape(N, _LAT // 4, 2, 2), (0, 2, 3, 1))
    sizes = [(2, 2)]
    for (_, _, k, s, op) in _DEC_CFG:
        h, w = sizes[-1]
        sizes.append((_convt_out(h, k, s, op), _convt_out(w, k, s, op)))
    nly = len(_DEC_CFG)
    wins = [None] * nly
    wins[-1] = ((_CROP_R[0] - 1, _CROP_R[1] + 1),
                (_CROP_C[0] - 1, _CROP_C[1] + 1))
    for li in range(nly - 1, 0, -1):
        (_, _, k, s, _) = _DEC_CFG[li]
        hin, win = sizes[li]
        wins[li - 1] = (_convt_need(wins[li][0], k, s, hin),
                        _convt_need(wins[li][1], k, s, win))
    in_off = (0, 0)
    for li, ((_, _, k, s, _), ly) in enumerate(zip(_DEC_CFG, dec)):
        hin, win = sizes[li]
        owr, owc = wins[li]
        x = _convt_window(x, in_off, ly["w"], k, s, hin, win, owr, owc,
                          ly["scale"], ly["bias"])
        in_off = (owr[0], owc[0])
    ones = jnp.ones((1, _NCLS), jnp.float32)
    y = _conv_direct(x, final_w, 3, ones, final_b, "linear",
                     out_dtype=jnp.float32)
    return jnp.transpose(y, (0, 3, 1, 2))


def _unpack(params):
    it = iter(params[0:150])
    blocks = []
    for (stride, has_exp, use_res) in _BLOCK_CFG:
        blk = {}
        blk["dw_s"], blk["dw_b"], blk["dw_w"] = next(it), next(it), next(it)
        if has_exp:
            blk["exp_s"], blk["exp_b"], blk["exp_w"] = \
                next(it), next(it), next(it)
        blk["proj_s"], blk["proj_b"], blk["proj_w"] = \
            next(it), next(it), next(it)
        blocks.append(blk)
    dec = []
    for li in range(6):
        b, s, w = params[150 + 3 * li: 153 + 3 * li]
        dec.append({"bias": b, "scale": s, "w": w})
    (final_b, final_w, head_s, head_b, head_w, mu_b, mu_w,
     post_b, post_w, stem_s, stem_b, stem_w, var_b, var_w) = params[168:182]
    return dict(
        blocks=blocks, dec=dec, final_b=final_b, final_w=final_w,
        head=(head_w, head_s, head_b), fc_mu=(mu_w, mu_b),
        post=(post_w, post_b), stem=(stem_w, stem_s, stem_b),
        fc_var=(var_w, var_b))


def kernel(x, eps, *params):
    P = _unpack(params)
    x_nhwc = jnp.transpose(x, (0, 2, 3, 1)).astype(jnp.float32)
    mu, log_var = _encode(x_nhwc, P["stem"], P["blocks"], P["head"],
                          P["post"], P["fc_mu"], P["fc_var"])
    z = _reparam(mu, log_var, eps)
    dec_params = [dict(w=d["w"], scale=d["scale"], bias=d["bias"])
                  for d in P["dec"]]
    recons = _decode(z, dec_params, P["final_w"], P["final_b"])
    return recons, x, mu, log_var


# speedup vs baseline: 9.7293x; 1.0302x over previous
import functools

import jax
import jax.numpy as jnp
from jax.experimental import pallas as pl
from jax.experimental.pallas import tpu as pltpu

_LAT = 64
_NCLS = 3
_SLOPE = 0.01
_CROP_R = (150, 234)
_CROP_C = (24, 360)

_BLOCK_CFG = [
    (1, False, False),
    (2, True, False), (1, True, True),
    (2, True, False), (1, True, True), (1, True, True),
    (2, True, False), (1, True, True), (1, True, True), (1, True, True),
    (1, True, False), (1, True, True), (1, True, True),
    (2, True, False), (1, True, True), (1, True, True),
    (1, True, False),
]


def _ru(x, m):
    return (x + m - 1) // m * m


def _act(y, act):
    if act == "relu6":
        return jnp.clip(y, 0.0, 6.0)
    if act == "leaky":
        return jnp.where(y >= 0.0, y, _SLOPE * y)
    if act == "relu":
        return jnp.maximum(y, 0.0)
    return y


def _mm_body(*refs, act, has_res):
    if has_res:
        a_ref, b_ref, s_ref, c_ref, r_ref, o_ref = refs
    else:
        a_ref, b_ref, s_ref, c_ref, o_ref = refs
    y = jnp.dot(a_ref[...], b_ref[...], preferred_element_type=jnp.float32)
    y = _act(y * s_ref[...] + c_ref[...], act)
    if has_res:
        y = y + r_ref[...].astype(jnp.float32)
    o_ref[...] = y.astype(o_ref.dtype)


def _mm_fused(a, b, scale, bias, act="linear", residual=None,
              out_dtype=jnp.bfloat16):
    M, K = a.shape
    N = b.shape[1]
    has_res = residual is not None
    per_row = 2 * K + N * (4 if out_dtype == jnp.float32 else 2)
    if has_res:
        per_row += 2 * N
    bm = max(8, min(4096, (2 * 1024 * 1024 // per_row) // 8 * 8))
    bm = min(bm, _ru(M, 8))
    Mp = _ru(M, bm)
    if Mp != M:
        a = jnp.pad(a, ((0, Mp - M), (0, 0)))
        if has_res:
            residual = jnp.pad(residual, ((0, Mp - M), (0, 0)))
    specs = [
        pl.BlockSpec((bm, K), lambda i: (i, 0)),
        pl.BlockSpec((K, N), lambda i: (0, 0)),
        pl.BlockSpec((1, N), lambda i: (0, 0)),
        pl.BlockSpec((1, N), lambda i: (0, 0)),
    ]
    ops = [a, b, scale, bias]
    if has_res:
        specs.append(pl.BlockSpec((bm, N), lambda i: (i, 0)))
        ops.append(residual)
    out = pl.pallas_call(
        functools.partial(_mm_body, act=act, has_res=has_res),
        out_shape=jax.ShapeDtypeStruct((Mp, N), out_dtype),
        grid=(Mp // bm,),
        in_specs=specs,
        out_specs=pl.BlockSpec((bm, N), lambda i: (i, 0)),
        compiler_params=pltpu.CompilerParams(
            dimension_semantics=("parallel",)),
    )(*ops)
    return out[:M] if Mp != M else out


def _im2col(x, kh, kw, stride, padding):
    if padding:
        x = jnp.pad(x, ((0, 0), (padding, padding), (padding, padding), (0, 0)))
    N, H, W, C = x.shape
    Ho = (H - kh) // stride + 1
    Wo = (W - kw) // stride + 1
    cols = [x[:, dy:dy + stride * Ho:stride, dx:dx + stride * Wo:stride, :]
            for dy in range(kh) for dx in range(kw)]
    patches = jnp.stack(cols, axis=3)
    return patches.reshape(N * Ho * Wo, kh * kw * C), (N, Ho, Wo)


def _conv(x, w, kh, kw, stride, padding, scale, bias, act,
          residual=None, out_dtype=jnp.bfloat16):
    x = x.astype(jnp.bfloat16)
    N, H, W, C = x.shape
    Cout = w.shape[1]
    if kh == 1 and kw == 1 and stride == 1 and padding == 0:
        patches, (N, Ho, Wo) = x.reshape(N * H * W, C), (N, H, W)
    else:
        patches, (N, Ho, Wo) = _im2col(x, kh, kw, stride, padding)
    res = residual.reshape(N * Ho * Wo, Cout) if residual is not None else None
    y = _mm_fused(patches, w, scale, bias, act, residual=res,
                  out_dtype=out_dtype)
    return y.reshape(N, Ho, Wo, Cout)


def _dw_s1_body(x_ref, w_ref, s_ref, c_ref, o_ref, *, Ho, Wo):
    xb = x_ref[0]
    H, W, C = xb.shape
    zr = jnp.zeros((1, W, C), xb.dtype)
    xp = jnp.concatenate([zr, xb, zr], axis=0)
    zc = jnp.zeros((H + 2, 1, C), xb.dtype)
    xp = jnp.concatenate([zc, xp, zc], axis=1)
    acc = None
    for dy in range(3):
        for dx in range(3):
            t = xp[dy:dy + Ho, dx:dx + Wo, :].astype(jnp.float32)
            term = t * w_ref[3 * dy + dx]
            acc = term if acc is None else acc + term
    y = jnp.clip(acc * s_ref[...] + c_ref[...], 0.0, 6.0)
    o_ref[0] = y.astype(jnp.bfloat16)


def _dw_s2_body(q0_ref, q1_ref, w_ref, s_ref, c_ref, o_ref, *, Ho, Wo, C):
    pb = (q0_ref[0, :, 0], q1_ref[0, :, 0])
    acc = None
    for dy in range(3):
        for dx in range(3):
            r = dx % 2
            ph = pb[dy % 2][:, :, r * C:(r + 1) * C]
            t = ph[dy // 2:dy // 2 + Ho, dx // 2:dx // 2 + Wo, :]
            term = t.astype(jnp.float32) * w_ref[3 * dy + dx]
            acc = term if acc is None else acc + term
    y = jnp.clip(acc * s_ref[...] + c_ref[...], 0.0, 6.0)
    o_ref[0] = y.astype(jnp.bfloat16)


def _depthwise(x, w9, scale, bias, stride):
    N, H, W, C = x.shape
    w = w9.reshape(9, C)
    wspec = pl.BlockSpec((9, C), lambda i: (0, 0))
    sspec = pl.BlockSpec((1, C), lambda i: (0, 0))
    if stride == 1:
        Ho, Wo = H, W
        out = pl.pallas_call(
            functools.partial(_dw_s1_body, Ho=Ho, Wo=Wo),
            out_shape=jax.ShapeDtypeStruct((N, Ho, Wo, C), jnp.bfloat16),
            grid=(N,),
            in_specs=[pl.BlockSpec((1, H, W, C), lambda i: (i, 0, 0, 0)),
                      wspec, sspec, sspec],
            out_specs=pl.BlockSpec((1, Ho, Wo, C), lambda i: (i, 0, 0, 0)),
            compiler_params=pltpu.CompilerParams(
                dimension_semantics=("parallel",)),
        )(x, w, scale, bias)
    else:
        xp = jnp.pad(x, ((0, 0), (1, 1), (1, 1), (0, 0)))
        Ho = (H + 2 - 3) // 2 + 1
        Wo = (W + 2 - 3) // 2 + 1
        Hp, Wp = H + 2, W + 2
        xv = xp.reshape(N, Hp // 2, 2, Wp // 2, 2 * C)
        qspec0 = pl.BlockSpec((1, Hp // 2, 1, Wp // 2, 2 * C),
                              lambda i: (i, 0, 0, 0, 0))
        qspec1 = pl.BlockSpec((1, Hp // 2, 1, Wp // 2, 2 * C),
                              lambda i: (i, 0, 1, 0, 0))
        out = pl.pallas_call(
            functools.partial(_dw_s2_body, Ho=Ho, Wo=Wo, C=C),
            out_shape=jax.ShapeDtypeStruct((N, Ho, Wo, C), jnp.bfloat16),
            grid=(N,),
            in_specs=[qspec0, qspec1, wspec, sspec, sspec],
            out_specs=pl.BlockSpec((1, Ho, Wo, C), lambda i: (i, 0, 0, 0)),
            compiler_params=pltpu.CompilerParams(
                dimension_semantics=("parallel",)),
        )(xv, xv, w, scale, bias)
    return out


def _stem_body(q0_ref, q1_ref, w_ref, s_ref, c_ref, o_ref, *, Ho, Wo, C):
    pb = (q0_ref[0, :, 0], q1_ref[0, :, 0])
    taps = []
    for dy in range(3):
        for dx in range(3):
            r = dx % 2
            ph = pb[dy % 2][:, :, r * C:(r + 1) * C]
            taps.append(ph[dy // 2:dy // 2 + Ho, dx // 2:dx // 2 + Wo, :])
    a = jnp.concatenate(taps, axis=-1).reshape(Ho * Wo, 9 * C)
    y = jnp.dot(a, w_ref[...], preferred_element_type=jnp.float32)
    y = jnp.clip(y * s_ref[...] + c_ref[...], 0.0, 6.0)
    o_ref[0] = y.reshape(Ho, Wo, o_ref.shape[3]).astype(jnp.bfloat16)


def _stem_conv(x, w, scale, bias):
    N, H, W, C = x.shape
    Cout = w.shape[1]
    Ho, Wo = H // 2, W // 2
    xp = jnp.pad(x.astype(jnp.bfloat16), ((0, 0), (1, 1), (1, 1), (0, 0)))
    Hp, Wp = H + 2, W + 2
    xv = xp.reshape(N, Hp // 2, 2, Wp // 2, 2 * C)
    qspec0 = pl.BlockSpec((1, Hp // 2, 1, Wp // 2, 2 * C),
                          lambda i: (i, 0, 0, 0, 0))
    qspec1 = pl.BlockSpec((1, Hp // 2, 1, Wp // 2, 2 * C),
                          lambda i: (i, 0, 1, 0, 0))
    return pl.pallas_call(
        functools.partial(_stem_body, Ho=Ho, Wo=Wo, C=C),
        out_shape=jax.ShapeDtypeStruct((N, Ho, Wo, Cout), jnp.bfloat16),
        grid=(N,),
        in_specs=[qspec0, qspec1,
                  pl.BlockSpec((9 * C, Cout), lambda i: (0, 0)),
                  pl.BlockSpec((1, Cout), lambda i: (0, 0)),
                  pl.BlockSpec((1, Cout), lambda i: (0, 0))],
        out_specs=pl.BlockSpec((1, Ho, Wo, Cout), lambda i: (i, 0, 0, 0)),
        compiler_params=pltpu.CompilerParams(
            dimension_semantics=("parallel",)),
    )(xv, xv, w, scale, bias)


def _convd_body(m_ref, h_ref, w_ref, s_ref, c_ref, o_ref, *,
                k, TR, Wo, act):
    C = m_ref.shape[3]
    xw = jnp.concatenate([m_ref[0], h_ref[0]], axis=0)
    taps = [xw[dy:dy + TR, dx:dx + Wo, :]
            for dy in range(k) for dx in range(k)]
    a = jnp.concatenate(taps, axis=-1).reshape(TR * Wo, k * k * C)
    y = jnp.dot(a, w_ref[...], preferred_element_type=jnp.float32)
    y = _act(y * s_ref[...] + c_ref[...], act)
    o_ref[0] = y.reshape(TR, Wo, o_ref.shape[3]).astype(o_ref.dtype)


def _conv_direct(x, w, k, scale, bias, act, out_dtype=jnp.bfloat16,
                 out_hw=None):
    x = x.astype(jnp.bfloat16)
    N, H, W, C = x.shape
    Cout = w.shape[1]
    Ho, Wo = out_hw if out_hw is not None else (H - k + 1, W - k + 1)
    Wop = _ru(Wo, 16)
    TR = min(32, _ru(Ho, 8))
    nt = -(-Ho // TR)
    xp = jnp.pad(x, ((0, 0), (0, max(0, TR * nt + 8 - H)),
                     (0, max(0, Wop + k - 1 - W)), (0, 0)))
    Lc = xp.shape[2]
    out = pl.pallas_call(
        functools.partial(_convd_body, k=k, TR=TR, Wo=Wop, act=act),
        out_shape=jax.ShapeDtypeStruct((N, TR * nt, Wop, Cout), out_dtype),
        grid=(N, nt),
        in_specs=[
            pl.BlockSpec((1, TR, Lc, C), lambda i, j: (i, j, 0, 0)),
            pl.BlockSpec((1, 8, Lc, C),
                         lambda i, j: (i, (j * TR + TR) // 8, 0, 0)),
            pl.BlockSpec((k * k * C, Cout), lambda i, j: (0, 0)),
            pl.BlockSpec((1, Cout), lambda i, j: (0, 0)),
            pl.BlockSpec((1, Cout), lambda i, j: (0, 0)),
        ],
        out_specs=pl.BlockSpec((1, TR, Wop, Cout), lambda i, j: (i, j, 0, 0)),
        compiler_params=pltpu.CompilerParams(
            dimension_semantics=("parallel", "parallel")),
    )(xp, xp, w, scale, bias)
    return out[:, :Ho, :Wo, :]


def _pool_body(x_ref, o_ref):
    m = jnp.mean(x_ref[...].astype(jnp.float32), axis=1)
    o_ref[...] = jnp.maximum(m, 0.0)


def _avgpool_relu(x):
    N, HW, C = x.shape
    return pl.pallas_call(
        _pool_body,
        out_shape=jax.ShapeDtypeStruct((N, C), jnp.float32),
    )(x)


def _passthrough_body(x_ref, o_ref):
    o_ref[...] = x_ref[...]


def _passthrough(x):
    N, C, H, W = x.shape
    xv = x.reshape(N, C * H, W)
    out = pl.pallas_call(
        _passthrough_body,
        out_shape=jax.ShapeDtypeStruct((N, C * H, W), x.dtype),
        grid=(N,),
        in_specs=[pl.BlockSpec((1, C * H, W), lambda i: (i, 0, 0))],
        out_specs=pl.BlockSpec((1, C * H, W), lambda i: (i, 0, 0)),
        compiler_params=pltpu.CompilerParams(
            dimension_semantics=("parallel",)),
    )(xv)
    return out.reshape(N, C, H, W)


def _reparam_body(mu_ref, lv_ref, eps_ref, o_ref):
    o_ref[...] = eps_ref[...] * jnp.exp(0.5 * lv_ref[...]) + mu_ref[...]


def _reparam(mu, log_var, eps):
    return pl.pallas_call(
        _reparam_body,
        out_shape=jax.ShapeDtypeStruct(mu.shape, jnp.float32),
    )(mu, log_var, eps)


def _cdiv(a, b):
    return -(-a // b)


def _convt_out(hin, k, s, op):
    return (hin - 1) * s + k + op


def _convt_need(win, k, s, hin):
    a, b = win
    d_last = (hin - 1) * s
    d_lo = max(a - (k - 1), 0)
    d_hi = min(b - 1, d_last)
    i_lo = max(0, min(_cdiv(d_lo, s), hin - 1))
    i_hi = max(i_lo, min(d_hi // s, hin - 1))
    return (i_lo, i_hi + 1)


def _convt_window(x, in_off, w, k, s, hin, win, owr, owc, scale, bias):
    N, xr, xc, C = x.shape
    ar, br = owr
    ac, bc = owc
    ir_lo, ir_hi = _convt_need(owr, k, s, hin)
    ic_lo, ic_hi = _convt_need(owc, k, s, win)
    xs = x[:, ir_lo - in_off[0]:ir_hi - in_off[0],
           ic_lo - in_off[1]:ic_hi - in_off[1], :].astype(jnp.bfloat16)
    nr, nc = ir_hi - ir_lo, ic_hi - ic_lo
    Lr = (br - ar) + k - 1
    Lc = (bc - ac) + k - 1
    sr = ir_lo * s + (k - 1) - ar
    sc = ic_lo * s + (k - 1) - ac
    buf = _dilate(xs, s, (sr, sc), (Lr, Lc))
    return _conv_direct(buf, w, k, scale, bias, "leaky",
                        out_hw=(br - ar, bc - ac))


def _dil_body(x_ref, o_ref, *, qs, r, C, s):
    q = pl.program_id(1)
    xb = x_ref[0]
    pads = [jnp.zeros(xb.shape[:2] + (r * C,), jnp.bfloat16),
            xb,
            jnp.zeros(xb.shape[:2] + ((s - 1 - r) * C,), jnp.bfloat16)]
    y = jnp.concatenate([p for p in pads if p.shape[-1]], axis=-1)
    o_ref[0, :, 0] = jnp.where(q == qs, y, jnp.zeros_like(y))


def _dilate(xs, s, off, size):
    N, nr, nc, C = xs.shape
    Lr, Lc = _ru(size[0], s), _ru(size[1], s)
    sr, sc = off
    xsp = jnp.pad(xs, ((0, 0),
                       (sr // s, Lr // s - sr // s - nr),
                       (sc // s, Lc // s - sc // s - nc),
                       (0, 0)))
    out = pl.pallas_call(
        functools.partial(_dil_body, qs=sr % s, r=sc % s, C=C, s=s),
        out_shape=jax.ShapeDtypeStruct((N, Lr // s, s, Lc // s, s * C),
                                       jnp.bfloat16),
        grid=(N, s),
        in_specs=[pl.BlockSpec((1, Lr // s, Lc // s, C),
                               lambda i, q: (i, 0, 0, 0))],
        out_specs=pl.BlockSpec((1, Lr // s, 1, Lc // s, s * C),
                               lambda i, q: (i, 0, q, 0, 0)),
        compiler_params=pltpu.CompilerParams(
            dimension_semantics=("parallel", "parallel")),
    )(xsp)
    return out.reshape(N, Lr, Lc, C)


def _encode(x_nhwc, stem, blocks, head, post, fc_mu, fc_var):
    stem_w, stem_s, stem_b = stem
    x = _stem_conv(x_nhwc, stem_w, stem_s, stem_b)
    for blk, (stride, has_exp, use_res) in zip(blocks, _BLOCK_CFG):
        inp = x
        h = x
        if has_exp:
            h = _conv(h, blk["exp_w"], 1, 1, 1, 0,
                      blk["exp_s"], blk["exp_b"], "relu6")
        h = _depthwise(h, blk["dw_w"], blk["dw_s"], blk["dw_b"], stride)
        x = _conv(h, blk["proj_w"], 1, 1, 1, 0,
                  blk["proj_s"], blk["proj_b"], "linear",
                  residual=inp if use_res else None)
    head_w, head_s, head_b = head
    x = _conv(x, head_w, 1, 1, 1, 0, head_s, head_b, "relu6")
    N, H, W, C = x.shape
    feat = _avgpool_relu(x.reshape(N, H * W, C))
    ones = jnp.ones((1, _LAT), jnp.float32)
    r = _mm_fused(feat.astype(jnp.bfloat16), post[0], ones, post[1],
                  "leaky", out_dtype=jnp.float32)
    mu = _mm_fused(r.astype(jnp.bfloat16), fc_mu[0], ones, fc_mu[1],
                   "linear", out_dtype=jnp.float32)
    log_var = _mm_fused(r.astype(jnp.bfloat16), fc_var[0], ones, fc_var[1],
                        "linear", out_dtype=jnp.float32)
    return mu, log_var


_DEC_CFG = [
    (_LAT // 4, 64, 5, 2, 0),
    (64, 64, 3, 2, 0),
    (64, 32, 3, 2, 0),
    (32, 16, 5, 3, 0),
    (16, 8, 3, 2, 0),
    (8, 8, 3, 2, 1),
]


def _decode(z, dec, final_w, final_b):
    N = z.shape[0]
    x = jnp.transpose(z.reshape(N, _LAT // 4, 2, 2), (0, 2, 3, 1))
    sizes = [(2, 2)]
    for (_, _, k, s, op) in _DEC_CFG:
        h, w = sizes[-1]
        sizes.append((_convt_out(h, k, s, op), _convt_out(w, k, s, op)))
    nly = len(_DEC_CFG)
    wins = [None] * nly
    wins[-1] = ((_CROP_R[0] - 1, _CROP_R[1] + 1),
                (_CROP_C[0] - 1, _CROP_C[1] + 1))
    for li in range(nly - 1, 0, -1):
        (_, _, k, s, _) = _DEC_CFG[li]
        hin, win = sizes[li]
        wins[li - 1] = (_convt_need(wins[li][0], k, s, hin),
                        _convt_need(wins[li][1], k, s, win))
    in_off = (0, 0)
    for li, ((_, _, k, s, _), ly) in enumerate(zip(_DEC_CFG, dec)):
        hin, win = sizes[li]
        owr, owc = wins[li]
        x = _convt_window(x, in_off, ly["w"], k, s, hin, win, owr, owc,
                          ly["scale"], ly["bias"])
        in_off = (owr[0], owc[0])
    ones = jnp.ones((1, _NCLS), jnp.float32)
    y = _conv_direct(x, final_w, 3, ones, final_b, "linear",
                     out_dtype=jnp.float32)
    return jnp.transpose(y, (0, 3, 1, 2))


def _unpack(params):
    it = iter(params[0:150])
    blocks = []
    for (stride, has_exp, use_res) in _BLOCK_CFG:
        blk = {}
        blk["dw_s"], blk["dw_b"], blk["dw_w"] = next(it), next(it), next(it)
        if has_exp:
            blk["exp_s"], blk["exp_b"], blk["exp_w"] = \
                next(it), next(it), next(it)
        blk["proj_s"], blk["proj_b"], blk["proj_w"] = \
            next(it), next(it), next(it)
        blocks.append(blk)
    dec = []
    for li in range(6):
        b, s, w = params[150 + 3 * li: 153 + 3 * li]
        dec.append({"bias": b, "scale": s, "w": w})
    (final_b, final_w, head_s, head_b, head_w, mu_b, mu_w,
     post_b, post_w, stem_s, stem_b, stem_w, var_b, var_w) = params[168:182]
    return dict(
        blocks=blocks, dec=dec, final_b=final_b, final_w=final_w,
        head=(head_w, head_s, head_b), fc_mu=(mu_w, mu_b),
        post=(post_w, post_b), stem=(stem_w, stem_s, stem_b),
        fc_var=(var_w, var_b))


def kernel(x, eps, *params):
    P = _unpack(params)
    x_nhwc = jnp.transpose(x, (0, 2, 3, 1)).astype(jnp.float32)
    mu, log_var = _encode(x_nhwc, P["stem"], P["blocks"], P["head"],
                          P["post"], P["fc_mu"], P["fc_var"])
    z = _reparam(mu, log_var, eps)
    dec_params = [dict(w=d["w"], scale=d["scale"], bias=d["bias"])
                  for d in P["dec"]]
    recons = _decode(z, dec_params, P["final_w"], P["final_b"])
    return recons, _passthrough(x), mu, log_var


# speedup vs baseline: 9.8069x; 1.0080x over previous
import functools

import jax
import jax.numpy as jnp
from jax.experimental import pallas as pl
from jax.experimental.pallas import tpu as pltpu

_LAT = 64
_NCLS = 3
_SLOPE = 0.01
_CROP_R = (150, 234)
_CROP_C = (24, 360)

_BLOCK_CFG = [
    (1, False, False),
    (2, True, False), (1, True, True),
    (2, True, False), (1, True, True), (1, True, True),
    (2, True, False), (1, True, True), (1, True, True), (1, True, True),
    (1, True, False), (1, True, True), (1, True, True),
    (2, True, False), (1, True, True), (1, True, True),
    (1, True, False),
]


def _ru(x, m):
    return (x + m - 1) // m * m


def _act(y, act):
    if act == "relu6":
        return jnp.clip(y, 0.0, 6.0)
    if act == "leaky":
        return jnp.where(y >= 0.0, y, _SLOPE * y)
    if act == "relu":
        return jnp.maximum(y, 0.0)
    return y


def _mm_body(*refs, act, has_res):
    if has_res:
        a_ref, b_ref, s_ref, c_ref, r_ref, o_ref = refs
    else:
        a_ref, b_ref, s_ref, c_ref, o_ref = refs
    y = jnp.dot(a_ref[...], b_ref[...], preferred_element_type=jnp.float32)
    y = _act(y * s_ref[...] + c_ref[...], act)
    if has_res:
        y = y + r_ref[...].astype(jnp.float32)
    o_ref[...] = y.astype(o_ref.dtype)


def _mm_fused(a, b, scale, bias, act="linear", residual=None,
              out_dtype=jnp.bfloat16):
    M, K = a.shape
    N = b.shape[1]
    has_res = residual is not None
    per_row = 2 * K + N * (4 if out_dtype == jnp.float32 else 2)
    if has_res:
        per_row += 2 * N
    bm = max(8, min(4096, (2 * 1024 * 1024 // per_row) // 8 * 8))
    bm = min(bm, _ru(M, 8))
    Mp = _ru(M, bm)
    if Mp != M:
        a = jnp.pad(a, ((0, Mp - M), (0, 0)))
        if has_res:
            residual = jnp.pad(residual, ((0, Mp - M), (0, 0)))
    specs = [
        pl.BlockSpec((bm, K), lambda i: (i, 0)),
        pl.BlockSpec((K, N), lambda i: (0, 0)),
        pl.BlockSpec((1, N), lambda i: (0, 0)),
        pl.BlockSpec((1, N), lambda i: (0, 0)),
    ]
    ops = [a, b, scale, bias]
    if has_res:
        specs.append(pl.BlockSpec((bm, N), lambda i: (i, 0)))
        ops.append(residual)
    out = pl.pallas_call(
        functools.partial(_mm_body, act=act, has_res=has_res),
        out_shape=jax.ShapeDtypeStruct((Mp, N), out_dtype),
        grid=(Mp // bm,),
        in_specs=specs,
        out_specs=pl.BlockSpec((bm, N), lambda i: (i, 0)),
        compiler_params=pltpu.CompilerParams(
            dimension_semantics=("parallel",)),
    )(*ops)
    return out[:M] if Mp != M else out


def _im2col(x, kh, kw, stride, padding):
    if padding:
        x = jnp.pad(x, ((0, 0), (padding, padding), (padding, padding), (0, 0)))
    N, H, W, C = x.shape
    Ho = (H - kh) // stride + 1
    Wo = (W - kw) // stride + 1
    cols = [x[:, dy:dy + stride * Ho:stride, dx:dx + stride * Wo:stride, :]
            for dy in range(kh) for dx in range(kw)]
    patches = jnp.stack(cols, axis=3)
    return patches.reshape(N * Ho * Wo, kh * kw * C), (N, Ho, Wo)


def _conv(x, w, kh, kw, stride, padding, scale, bias, act,
          residual=None, out_dtype=jnp.bfloat16):
    x = x.astype(jnp.bfloat16)
    N, H, W, C = x.shape
    Cout = w.shape[1]
    if kh == 1 and kw == 1 and stride == 1 and padding == 0:
        patches, (N, Ho, Wo) = x.reshape(N * H * W, C), (N, H, W)
    else:
        patches, (N, Ho, Wo) = _im2col(x, kh, kw, stride, padding)
    res = residual.reshape(N * Ho * Wo, Cout) if residual is not None else None
    y = _mm_fused(patches, w, scale, bias, act, residual=res,
                  out_dtype=out_dtype)
    return y.reshape(N, Ho, Wo, Cout)


def _dw_s1_body(x_ref, w_ref, s_ref, c_ref, o_ref, *, Ho, Wo):
    xb = x_ref[0]
    H, W, C = xb.shape
    zr = jnp.zeros((1, W, C), xb.dtype)
    xp = jnp.concatenate([zr, xb, zr], axis=0)
    zc = jnp.zeros((H + 2, 1, C), xb.dtype)
    xp = jnp.concatenate([zc, xp, zc], axis=1)
    acc = None
    for dy in range(3):
        for dx in range(3):
            t = xp[dy:dy + Ho, dx:dx + Wo, :].astype(jnp.float32)
            term = t * w_ref[3 * dy + dx]
            acc = term if acc is None else acc + term
    y = jnp.clip(acc * s_ref[...] + c_ref[...], 0.0, 6.0)
    o_ref[0] = y.astype(jnp.bfloat16)


def _dw_s2_body(q0_ref, q1_ref, w_ref, s_ref, c_ref, o_ref, *, Ho, Wo, C):
    pb = (q0_ref[0, :, 0], q1_ref[0, :, 0])
    acc = None
    for dy in range(3):
        for dx in range(3):
            r = dx % 2
            ph = pb[dy % 2][:, :, r * C:(r + 1) * C]
            t = ph[dy // 2:dy // 2 + Ho, dx // 2:dx // 2 + Wo, :]
            term = t.astype(jnp.float32) * w_ref[3 * dy + dx]
            acc = term if acc is None else acc + term
    y = jnp.clip(acc * s_ref[...] + c_ref[...], 0.0, 6.0)
    o_ref[0] = y.astype(jnp.bfloat16)


def _depthwise(x, w9, scale, bias, stride):
    N, H, W, C = x.shape
    w = w9.reshape(9, C)
    wspec = pl.BlockSpec((9, C), lambda i: (0, 0))
    sspec = pl.BlockSpec((1, C), lambda i: (0, 0))
    if stride == 1:
        Ho, Wo = H, W
        out = pl.pallas_call(
            functools.partial(_dw_s1_body, Ho=Ho, Wo=Wo),
            out_shape=jax.ShapeDtypeStruct((N, Ho, Wo, C), jnp.bfloat16),
            grid=(N,),
            in_specs=[pl.BlockSpec((1, H, W, C), lambda i: (i, 0, 0, 0)),
                      wspec, sspec, sspec],
            out_specs=pl.BlockSpec((1, Ho, Wo, C), lambda i: (i, 0, 0, 0)),
            compiler_params=pltpu.CompilerParams(
                dimension_semantics=("parallel",)),
        )(x, w, scale, bias)
    else:
        xp = jnp.pad(x, ((0, 0), (1, 1), (1, 1), (0, 0)))
        Ho = (H + 2 - 3) // 2 + 1
        Wo = (W + 2 - 3) // 2 + 1
        Hp, Wp = H + 2, W + 2
        xv = xp.reshape(N, Hp // 2, 2, Wp // 2, 2 * C)
        qspec0 = pl.BlockSpec((1, Hp // 2, 1, Wp // 2, 2 * C),
                              lambda i: (i, 0, 0, 0, 0))
        qspec1 = pl.BlockSpec((1, Hp // 2, 1, Wp // 2, 2 * C),
                              lambda i: (i, 0, 1, 0, 0))
        out = pl.pallas_call(
            functools.partial(_dw_s2_body, Ho=Ho, Wo=Wo, C=C),
            out_shape=jax.ShapeDtypeStruct((N, Ho, Wo, C), jnp.bfloat16),
            grid=(N,),
            in_specs=[qspec0, qspec1, wspec, sspec, sspec],
            out_specs=pl.BlockSpec((1, Ho, Wo, C), lambda i: (i, 0, 0, 0)),
            compiler_params=pltpu.CompilerParams(
                dimension_semantics=("parallel",)),
        )(xv, xv, w, scale, bias)
    return out


def _stem_body(q0_ref, q1_ref, w_ref, s_ref, c_ref, o_ref, *, Ho, Wo, C):
    pb = (q0_ref[0, :, 0], q1_ref[0, :, 0])
    taps = []
    for dy in range(3):
        for dx in range(3):
            r = dx % 2
            ph = pb[dy % 2][:, :, r * C:(r + 1) * C]
            taps.append(ph[dy // 2:dy // 2 + Ho, dx // 2:dx // 2 + Wo, :])
    a = jnp.concatenate(taps, axis=-1).reshape(Ho * Wo, 9 * C)
    y = jnp.dot(a, w_ref[...], preferred_element_type=jnp.float32)
    y = jnp.clip(y * s_ref[...] + c_ref[...], 0.0, 6.0)
    o_ref[0] = y.reshape(Ho, Wo, o_ref.shape[3]).astype(jnp.bfloat16)


def _stem_conv(x, w, scale, bias):
    N, H, W, C = x.shape
    Cout = w.shape[1]
    Ho, Wo = H // 2, W // 2
    xp = jnp.pad(x.astype(jnp.bfloat16), ((0, 0), (1, 1), (1, 1), (0, 0)))
    Hp, Wp = H + 2, W + 2
    xv = xp.reshape(N, Hp // 2, 2, Wp // 2, 2 * C)
    qspec0 = pl.BlockSpec((1, Hp // 2, 1, Wp // 2, 2 * C),
                          lambda i: (i, 0, 0, 0, 0))
    qspec1 = pl.BlockSpec((1, Hp // 2, 1, Wp // 2, 2 * C),
                          lambda i: (i, 0, 1, 0, 0))
    return pl.pallas_call(
        functools.partial(_stem_body, Ho=Ho, Wo=Wo, C=C),
        out_shape=jax.ShapeDtypeStruct((N, Ho, Wo, Cout), jnp.bfloat16),
        grid=(N,),
        in_specs=[qspec0, qspec1,
                  pl.BlockSpec((9 * C, Cout), lambda i: (0, 0)),
                  pl.BlockSpec((1, Cout), lambda i: (0, 0)),
                  pl.BlockSpec((1, Cout), lambda i: (0, 0))],
        out_specs=pl.BlockSpec((1, Ho, Wo, Cout), lambda i: (i, 0, 0, 0)),
        compiler_params=pltpu.CompilerParams(
            dimension_semantics=("parallel",)),
    )(xv, xv, w, scale, bias)


def _convd_body(m_ref, h_ref, w_ref, s_ref, c_ref, o_ref, *,
                k, TR, Wo, act):
    C = m_ref.shape[3]
    xw = jnp.concatenate([m_ref[0], h_ref[0]], axis=0)
    taps = [xw[dy:dy + TR, dx:dx + Wo, :]
            for dy in range(k) for dx in range(k)]
    a = jnp.concatenate(taps, axis=-1).reshape(TR * Wo, k * k * C)
    y = jnp.dot(a, w_ref[...], preferred_element_type=jnp.float32)
    y = _act(y * s_ref[...] + c_ref[...], act)
    o_ref[0] = y.reshape(TR, Wo, o_ref.shape[3]).astype(o_ref.dtype)


def _conv_direct(x, w, k, scale, bias, act, out_dtype=jnp.bfloat16,
                 out_hw=None):
    x = x.astype(jnp.bfloat16)
    N, H, W, C = x.shape
    Cout = w.shape[1]
    Ho, Wo = out_hw if out_hw is not None else (H - k + 1, W - k + 1)
    Wop = _ru(Wo, 16)
    TR = min(48, _ru(Ho, 8))
    nt = -(-Ho // TR)
    xp = jnp.pad(x, ((0, 0), (0, max(0, TR * nt + 8 - H)),
                     (0, max(0, Wop + k - 1 - W)), (0, 0)))
    Lc = xp.shape[2]
    out = pl.pallas_call(
        functools.partial(_convd_body, k=k, TR=TR, Wo=Wop, act=act),
        out_shape=jax.ShapeDtypeStruct((N, TR * nt, Wop, Cout), out_dtype),
        grid=(N, nt),
        in_specs=[
            pl.BlockSpec((1, TR, Lc, C), lambda i, j: (i, j, 0, 0)),
            pl.BlockSpec((1, 8, Lc, C),
                         lambda i, j: (i, (j * TR + TR) // 8, 0, 0)),
            pl.BlockSpec((k * k * C, Cout), lambda i, j: (0, 0)),
            pl.BlockSpec((1, Cout), lambda i, j: (0, 0)),
            pl.BlockSpec((1, Cout), lambda i, j: (0, 0)),
        ],
        out_specs=pl.BlockSpec((1, TR, Wop, Cout), lambda i, j: (i, j, 0, 0)),
        compiler_params=pltpu.CompilerParams(
            dimension_semantics=("parallel", "parallel")),
    )(xp, xp, w, scale, bias)
    return out[:, :Ho, :Wo, :]


def _pool_body(x_ref, o_ref):
    m = jnp.mean(x_ref[...].astype(jnp.float32), axis=1)
    o_ref[...] = jnp.maximum(m, 0.0)


def _avgpool_relu(x):
    N, HW, C = x.shape
    return pl.pallas_call(
        _pool_body,
        out_shape=jax.ShapeDtypeStruct((N, C), jnp.float32),
    )(x)


def _passthrough_body(x_ref, o_ref):
    o_ref[...] = x_ref[...]


def _passthrough(x):
    N, C, H, W = x.shape
    xv = x.reshape(N, C * H, W)
    out = pl.pallas_call(
        _passthrough_body,
        out_shape=jax.ShapeDtypeStruct((N, C * H, W), x.dtype),
        grid=(N,),
        in_specs=[pl.BlockSpec((1, C * H, W), lambda i: (i, 0, 0))],
        out_specs=pl.BlockSpec((1, C * H, W), lambda i: (i, 0, 0)),
        compiler_params=pltpu.CompilerParams(
            dimension_semantics=("parallel",)),
    )(xv)
    return out.reshape(N, C, H, W)


def _reparam_body(mu_ref, lv_ref, eps_ref, o_ref):
    o_ref[...] = eps_ref[...] * jnp.exp(0.5 * lv_ref[...]) + mu_ref[...]


def _reparam(mu, log_var, eps):
    return pl.pallas_call(
        _reparam_body,
        out_shape=jax.ShapeDtypeStruct(mu.shape, jnp.float32),
    )(mu, log_var, eps)


def _cdiv(a, b):
    return -(-a // b)


def _convt_out(hin, k, s, op):
    return (hin - 1) * s + k + op


def _convt_need(win, k, s, hin):
    a, b = win
    d_last = (hin - 1) * s
    d_lo = max(a - (k - 1), 0)
    d_hi = min(b - 1, d_last)
    i_lo = max(0, min(_cdiv(d_lo, s), hin - 1))
    i_hi = max(i_lo, min(d_hi // s, hin - 1))
    return (i_lo, i_hi + 1)


def _convt_window(x, in_off, w, k, s, hin, win, owr, owc, scale, bias):
    N, xr, xc, C = x.shape
    ar, br = owr
    ac, bc = owc
    ir_lo, ir_hi = _convt_need(owr, k, s, hin)
    ic_lo, ic_hi = _convt_need(owc, k, s, win)
    xs = x[:, ir_lo - in_off[0]:ir_hi - in_off[0],
           ic_lo - in_off[1]:ic_hi - in_off[1], :].astype(jnp.bfloat16)
    nr, nc = ir_hi - ir_lo, ic_hi - ic_lo
    Lr = (br - ar) + k - 1
    Lc = (bc - ac) + k - 1
    sr = ir_lo * s + (k - 1) - ar
    sc = ic_lo * s + (k - 1) - ac
    buf = _dilate(xs, s, (sr, sc), (Lr, Lc))
    return _conv_direct(buf, w, k, scale, bias, "leaky",
                        out_hw=(br - ar, bc - ac))


def _dil_body(x_ref, o_ref, *, qs, r, C, s):
    q = pl.program_id(1)
    xb = x_ref[0]
    pads = [jnp.zeros(xb.shape[:2] + (r * C,), jnp.bfloat16),
            xb,
            jnp.zeros(xb.shape[:2] + ((s - 1 - r) * C,), jnp.bfloat16)]
    y = jnp.concatenate([p for p in pads if p.shape[-1]], axis=-1)
    o_ref[0, :, 0] = jnp.where(q == qs, y, jnp.zeros_like(y))


def _dilate(xs, s, off, size):
    N, nr, nc, C = xs.shape
    Lr, Lc = _ru(size[0], s), _ru(size[1], s)
    sr, sc = off
    xsp = jnp.pad(xs, ((0, 0),
                       (sr // s, Lr // s - sr // s - nr),
                       (sc // s, Lc // s - sc // s - nc),
                       (0, 0)))
    out = pl.pallas_call(
        functools.partial(_dil_body, qs=sr % s, r=sc % s, C=C, s=s),
        out_shape=jax.ShapeDtypeStruct((N, Lr // s, s, Lc // s, s * C),
                                       jnp.bfloat16),
        grid=(N, s),
        in_specs=[pl.BlockSpec((1, Lr // s, Lc // s, C),
                               lambda i, q: (i, 0, 0, 0))],
        out_specs=pl.BlockSpec((1, Lr // s, 1, Lc // s, s * C),
                               lambda i, q: (i, 0, q, 0, 0)),
        compiler_params=pltpu.CompilerParams(
            dimension_semantics=("parallel", "parallel")),
    )(xsp)
    return out.reshape(N, Lr, Lc, C)


def _encode(x_nhwc, stem, blocks, head, post, fc_mu, fc_var):
    stem_w, stem_s, stem_b = stem
    x = _stem_conv(x_nhwc, stem_w, stem_s, stem_b)
    for blk, (stride, has_exp, use_res) in zip(blocks, _BLOCK_CFG):
        inp = x
        h = x
        if has_exp:
            h = _conv(h, blk["exp_w"], 1, 1, 1, 0,
                      blk["exp_s"], blk["exp_b"], "relu6")
        h = _depthwise(h, blk["dw_w"], blk["dw_s"], blk["dw_b"], stride)
        x = _conv(h, blk["proj_w"], 1, 1, 1, 0,
                  blk["proj_s"], blk["proj_b"], "linear",
                  residual=inp if use_res else None)
    head_w, head_s, head_b = head
    x = _conv(x, head_w, 1, 1, 1, 0, head_s, head_b, "relu6")
    N, H, W, C = x.shape
    feat = _avgpool_relu(x.reshape(N, H * W, C))
    ones = jnp.ones((1, _LAT), jnp.float32)
    r = _mm_fused(feat.astype(jnp.bfloat16), post[0], ones, post[1],
                  "leaky", out_dtype=jnp.float32)
    mu = _mm_fused(r.astype(jnp.bfloat16), fc_mu[0], ones, fc_mu[1],
                   "linear", out_dtype=jnp.float32)
    log_var = _mm_fused(r.astype(jnp.bfloat16), fc_var[0], ones, fc_var[1],
                        "linear", out_dtype=jnp.float32)
    return mu, log_var


_DEC_CFG = [
    (_LAT // 4, 64, 5, 2, 0),
    (64, 64, 3, 2, 0),
    (64, 32, 3, 2, 0),
    (32, 16, 5, 3, 0),
    (16, 8, 3, 2, 0),
    (8, 8, 3, 2, 1),
]


def _decode(z, dec, final_w, final_b):
    N = z.shape[0]
    x = jnp.transpose(z.reshape(N, _LAT // 4, 2, 2), (0, 2, 3, 1))
    sizes = [(2, 2)]
    for (_, _, k, s, op) in _DEC_CFG:
        h, w = sizes[-1]
        sizes.append((_convt_out(h, k, s, op), _convt_out(w, k, s, op)))
    nly = len(_DEC_CFG)
    wins = [None] * nly
    wins[-1] = ((_CROP_R[0] - 1, _CROP_R[1] + 1),
                (_CROP_C[0] - 1, _CROP_C[1] + 1))
    for li in range(nly - 1, 0, -1):
        (_, _, k, s, _) = _DEC_CFG[li]
        hin, win = sizes[li]
        wins[li - 1] = (_convt_need(wins[li][0], k, s, hin),
                        _convt_need(wins[li][1], k, s, win))
    in_off = (0, 0)
    for li, ((_, _, k, s, _), ly) in enumerate(zip(_DEC_CFG, dec)):
        hin, win = sizes[li]
        owr, owc = wins[li]
        x = _convt_window(x, in_off, ly["w"], k, s, hin, win, owr, owc,
                          ly["scale"], ly["bias"])
        in_off = (owr[0], owc[0])
    ones = jnp.ones((1, _NCLS), jnp.float32)
    y = _conv_direct(x, final_w, 3, ones, final_b, "linear",
                     out_dtype=jnp.float32)
    return jnp.transpose(y, (0, 3, 1, 2))


def _unpack(params):
    it = iter(params[0:150])
    blocks = []
    for (stride, has_exp, use_res) in _BLOCK_CFG:
        blk = {}
        blk["dw_s"], blk["dw_b"], blk["dw_w"] = next(it), next(it), next(it)
        if has_exp:
            blk["exp_s"], blk["exp_b"], blk["exp_w"] = \
                next(it), next(it), next(it)
        blk["proj_s"], blk["proj_b"], blk["proj_w"] = \
            next(it), next(it), next(it)
        blocks.append(blk)
    dec = []
    for li in range(6):
        b, s, w = params[150 + 3 * li: 153 + 3 * li]
        dec.append({"bias": b, "scale": s, "w": w})
    (final_b, final_w, head_s, head_b, head_w, mu_b, mu_w,
     post_b, post_w, stem_s, stem_b, stem_w, var_b, var_w) = params[168:182]
    return dict(
        blocks=blocks, dec=dec, final_b=final_b, final_w=final_w,
        head=(head_w, head_s, head_b), fc_mu=(mu_w, mu_b),
        post=(post_w, post_b), stem=(stem_w, stem_s, stem_b),
        fc_var=(var_w, var_b))


def kernel(x, eps, *params):
    P = _unpack(params)
    x_nhwc = jnp.transpose(x, (0, 2, 3, 1)).astype(jnp.float32)
    mu, log_var = _encode(x_nhwc, P["stem"], P["blocks"], P["head"],
                          P["post"], P["fc_mu"], P["fc_var"])
    z = _reparam(mu, log_var, eps)
    dec_params = [dict(w=d["w"], scale=d["scale"], bias=d["bias"])
                  for d in P["dec"]]
    recons = _decode(z, dec_params, P["final_w"], P["final_b"])
    return recons, _passthrough(x), mu, log_var
